# baseline (speedup 1.0000x reference)
import functools

import jax
import jax.numpy as jnp
from jax import lax
from jax.experimental import pallas as pl
from jax.experimental.pallas import tpu as pltpu
from jax.experimental.pallas import tpu_sc as plsc

N_HOST = 10000
N_FLOW = 100000
E = 600000
D_IN = 128
D_H = 64

CHUNK = 16384
K_CHUNKS = -(-N_FLOW // CHUNK)
P_FLOW = K_CHUNKS * CHUNK
P_HOST = 10240


def _proj_body(x_ref, w_ref, b_ref, o_ref):
    y = jnp.dot(x_ref[...], w_ref[...], preferred_element_type=jnp.float32)
    o_ref[...] = jnp.maximum(y + b_ref[...], 0.0)


def _tc_proj_relu(x, w, b, rb):
    n = x.shape[0]
    return pl.pallas_call(
        _proj_body,
        grid=(n // rb,),
        in_specs=[
            pl.BlockSpec((rb, D_IN), lambda i: (i, 0)),
            pl.BlockSpec((D_IN, D_H), lambda i: (0, 0)),
            pl.BlockSpec((1, D_H), lambda i: (0, 0)),
        ],
        out_specs=pl.BlockSpec((rb, D_H), lambda i: (i, 0)),
        out_shape=jax.ShapeDtypeStruct((n, D_H), jnp.float32),
    )(x, w, b.reshape(1, D_H))


def _layer_body(parts_ref, cnt_ref, prev_ref, wl_ref, bl_ref, wr_ref, o_ref):
    s = parts_ref[0] + parts_ref[1]
    c = jnp.maximum(cnt_ref[0, :, 0:1] + cnt_ref[1, :, 0:1], 1.0)
    agg = s / c
    y = (jnp.dot(agg, wl_ref[...], preferred_element_type=jnp.float32)
         + bl_ref[...]
         + jnp.dot(prev_ref[...], wr_ref[...], preferred_element_type=jnp.float32))
    o_ref[...] = jnp.where(y >= 0.0, y, 0.01 * y)


def _tc_layer(parts, cnt, prev, wl, bl, wr, rb):
    n = prev.shape[0]
    return pl.pallas_call(
        _layer_body,
        grid=(n // rb,),
        in_specs=[
            pl.BlockSpec((2, rb, D_H), lambda i: (0, i, 0)),
            pl.BlockSpec((2, rb, _CW), lambda i: (0, i, 0)),
            pl.BlockSpec((rb, D_H), lambda i: (i, 0)),
            pl.BlockSpec((D_H, D_H), lambda i: (0, 0)),
            pl.BlockSpec((1, D_H), lambda i: (0, 0)),
            pl.BlockSpec((D_H, D_H), lambda i: (0, 0)),
        ],
        out_specs=pl.BlockSpec((rb, D_H), lambda i: (i, 0)),
        out_shape=jax.ShapeDtypeStruct((n, D_H), jnp.float32),
    )(parts, cnt, prev, wl, bl.reshape(1, D_H), wr)


def _final_body(parts_ref, cnt_ref, prev_ref, wl_ref, bl_ref, wr_ref,
                wo_ref, bo_ref, o_ref):
    s = parts_ref[0] + parts_ref[1]
    c = jnp.maximum(cnt_ref[0, :, 0:1] + cnt_ref[1, :, 0:1], 1.0)
    agg = s / c
    y = (jnp.dot(agg, wl_ref[...], preferred_element_type=jnp.float32)
         + bl_ref[...]
         + jnp.dot(prev_ref[...], wr_ref[...], preferred_element_type=jnp.float32))
    f2 = jnp.where(y >= 0.0, y, 0.01 * y)
    o_ref[...] = (jnp.dot(f2, wo_ref[...], preferred_element_type=jnp.float32)
                  + bo_ref[...])


def _tc_final(parts, cnt, prev, wl, bl, wr, wo, bo, rb):
    n = prev.shape[0]
    d_out = wo.shape[1]
    return pl.pallas_call(
        _final_body,
        grid=(n // rb,),
        in_specs=[
            pl.BlockSpec((2, rb, D_H), lambda i: (0, i, 0)),
            pl.BlockSpec((2, rb, _CW), lambda i: (0, i, 0)),
            pl.BlockSpec((rb, D_H), lambda i: (i, 0)),
            pl.BlockSpec((D_H, D_H), lambda i: (0, 0)),
            pl.BlockSpec((1, D_H), lambda i: (0, 0)),
            pl.BlockSpec((D_H, D_H), lambda i: (0, 0)),
            pl.BlockSpec((D_H, d_out), lambda i: (0, 0)),
            pl.BlockSpec((1, d_out), lambda i: (0, 0)),
        ],
        out_specs=pl.BlockSpec((rb, d_out), lambda i: (i, 0)),
        out_shape=jax.ShapeDtypeStruct((n, d_out), jnp.float32),
    )(parts, cnt, prev, wl, bl.reshape(1, D_H), wr, wo, bo.reshape(1, d_out))


_NC, _NS = 2, 16
_NW = _NC * _NS
_CW = 8

_BH = 960
_HSTEPS = E // _BH


def _sc_agg_host_body(tab, src, dst, ones_hbm, z64, zc,
                      out_agg, out_cnt,
                      src_v, dst_v, rows_v, ones_v, agg_sh, cnt_sh,
                      sem, sem2, sem3):
    c = lax.axis_index("c")
    s = lax.axis_index("s")
    wid = s * _NC + c
    rp = P_HOST // _NS
    pltpu.sync_copy(ones_hbm, ones_v)
    pltpu.sync_copy(z64.at[pl.ds(s * rp, rp)], agg_sh.at[pl.ds(s * rp, rp)])
    pltpu.sync_copy(zc.at[pl.ds(s * rp, rp)], cnt_sh.at[pl.ds(s * rp, rp)])
    plsc.subcore_barrier()

    def step(i, carry):
        chunk = i * _NW + wid

        @pl.when(chunk < _HSTEPS)
        def _():
            base = chunk * _BH
            d1 = pltpu.async_copy(src.at[pl.ds(base, _BH)], src_v, sem)
            d2 = pltpu.async_copy(dst.at[pl.ds(base, _BH)], dst_v, sem2)
            d1.wait()
            g = pltpu.async_copy(tab.at[src_v], rows_v, sem3)
            d2.wait()
            g.wait()
            s1 = pltpu.async_copy(rows_v, agg_sh.at[dst_v], sem, add=True)
            pltpu.async_copy(ones_v, cnt_sh.at[dst_v], sem2, add=True).wait()
            s1.wait()

        return carry

    lax.fori_loop(0, (_HSTEPS + _NW - 1) // _NW, step, 0)
    plsc.subcore_barrier()
    pltpu.sync_copy(agg_sh.at[pl.ds(s * rp, rp)],
                    out_agg.at[c, pl.ds(s * rp, rp)])
    pltpu.sync_copy(cnt_sh.at[pl.ds(s * rp, rp)],
                    out_cnt.at[c, pl.ds(s * rp, rp)])


def _sc_agg_host(f0, src, dst):
    mesh = plsc.VectorSubcoreMesh(core_axis_name="c", subcore_axis_name="s")
    ones = jnp.ones((_BH, _CW), jnp.float32)
    z64 = jnp.zeros((P_HOST, D_H), jnp.float32)
    zc = jnp.zeros((P_HOST, _CW), jnp.float32)
    f = pl.kernel(
        _sc_agg_host_body,
        out_type=(jax.ShapeDtypeStruct((_NC, P_HOST, D_H), jnp.float32),
                  jax.ShapeDtypeStruct((_NC, P_HOST, _CW), jnp.float32)),
        mesh=mesh,
        scratch_types=[
            pltpu.VMEM((_BH,), jnp.int32),
            pltpu.VMEM((_BH,), jnp.int32),
            pltpu.VMEM((_BH, D_H), jnp.float32),
            pltpu.VMEM((_BH, _CW), jnp.float32),
            pltpu.VMEM_SHARED((P_HOST, D_H), jnp.float32),
            pltpu.VMEM_SHARED((P_HOST, _CW), jnp.float32),
            pltpu.SemaphoreType.DMA,
            pltpu.SemaphoreType.DMA,
            pltpu.SemaphoreType.DMA,
        ],
        compiler_params=pltpu.CompilerParams(use_tc_tiling_on_sc=False),
    )
    return f(f0, src, dst, ones, z64, zc)


_SHIFT = 14
_MASK = CHUNK - 1
_DUMP = CHUNK
_BE = 960
_NCH = E // _BE
_F = 256
_CAPU = 19456
_B2 = 128


def _sc_bucket_body(src, dst, out_src, out_dst, out_cnt,
                    src_c, dst_c, buf_s, buf_d, cnt_v):
    c = lax.axis_index("c")
    s = lax.axis_index("s")
    wid = s * _NC + c
    iota = lax.iota(jnp.int32, 16)
    base_trips = _NCH // _NW
    trips = jnp.where(wid < _NCH - base_trips * _NW, base_trips + 1, base_trips)

    def chunk_step(i, carry):
        ch = i * _NW + wid
        base = ch * _BE
        pltpu.sync_copy(src.at[pl.ds(base, _BE)], src_c)
        pltpu.sync_copy(dst.at[pl.ds(base, _BE)], dst_c)

        def vreg_step(v, carry2):
            ptrs, fcs = carry2
            s16 = src_c[pl.ds(v * 16, 16)]
            d16 = dst_c[pl.ds(v * 16, 16)]
            k16 = lax.shift_right_logical(d16, _SHIFT)
            dl16 = lax.bitwise_and(d16, _MASK)
            new_ptrs, new_fcs = [], []
            for k0 in range(K_CHUNKS):
                p, fc = ptrs[k0], fcs[k0]
                m = k16 == k0
                n = jnp.sum(m.astype(jnp.int32))
                plsc.store_compressed(buf_s.at[pl.ds(k0 * 512 + p, 16)], s16, mask=m)
                plsc.store_compressed(buf_d.at[pl.ds(k0 * 512 + p, 16)], dl16, mask=m)
                p = p + n
                full = p >= _F

                @pl.when(full)
                def _(k0=k0, fc=fc):
                    pltpu.sync_copy(buf_s.at[pl.ds(k0 * 512, _F)],
                                    out_src.at[k0, wid, pl.ds(fc * _F, _F)])
                    pltpu.sync_copy(buf_d.at[pl.ds(k0 * 512, _F)],
                                    out_dst.at[k0, wid, pl.ds(fc * _F, _F)])
                    buf_s[pl.ds(k0 * 512, 16)] = buf_s[pl.ds(k0 * 512 + _F, 16)]
                    buf_d[pl.ds(k0 * 512, 16)] = buf_d[pl.ds(k0 * 512 + _F, 16)]

                new_ptrs.append(jnp.where(full, p - _F, p))
                new_fcs.append(jnp.where(full, fc + 1, fc))
            return tuple(new_ptrs), tuple(new_fcs)

        return lax.fori_loop(0, _BE // 16, vreg_step, carry)

    zero = jnp.int32(0)
    ptrs, fcs = lax.fori_loop(
        0, trips, chunk_step,
        (tuple(zero for _ in range(K_CHUNKS)),
         tuple(zero for _ in range(K_CHUNKS))))

    cnts = jnp.zeros((16,), jnp.int32)
    for k0 in range(K_CHUNKS):
        p, fc = ptrs[k0], fcs[k0]
        buf_s[pl.ds(k0 * 512 + p, 16)] = iota
        buf_d[pl.ds(k0 * 512 + p, 16)] = jnp.full((16,), _DUMP, jnp.int32)
        n16 = lax.shift_right_logical(p + 15, 4)

        def fl(j, _, k0=k0, fc=fc):
            pltpu.sync_copy(buf_s.at[pl.ds(k0 * 512 + j * 16, 16)],
                            out_src.at[k0, wid, pl.ds(fc * _F + j * 16, 16)])
            pltpu.sync_copy(buf_d.at[pl.ds(k0 * 512 + j * 16, 16)],
                            out_dst.at[k0, wid, pl.ds(fc * _F + j * 16, 16)])
            return _

        lax.fori_loop(0, n16, fl, 0)
        cnts = cnts + jnp.where(iota == k0, fc * (_F // 16) + n16, 0)
    cnt_v[...] = cnts
    pltpu.sync_copy(cnt_v, out_cnt.at[wid])


def _sc_bucket(src, dst):
    mesh = plsc.VectorSubcoreMesh(core_axis_name="c", subcore_axis_name="s")
    f = pl.kernel(
        _sc_bucket_body,
        out_type=(jax.ShapeDtypeStruct((K_CHUNKS, _NW, _CAPU), jnp.int32),
                  jax.ShapeDtypeStruct((K_CHUNKS, _NW, _CAPU), jnp.int32),
                  jax.ShapeDtypeStruct((_NW, 16), jnp.int32)),
        mesh=mesh,
        scratch_types=[
            pltpu.VMEM((_BE,), jnp.int32),
            pltpu.VMEM((_BE,), jnp.int32),
            pltpu.VMEM((K_CHUNKS * 512,), jnp.int32),
            pltpu.VMEM((K_CHUNKS * 512,), jnp.int32),
            pltpu.VMEM((16,), jnp.int32),
        ],
        compiler_params=pltpu.CompilerParams(use_tc_tiling_on_sc=False,
                                             needs_layout_passes=False),
    )
    return f(src, dst)


def _make_sc_agg_flow(with_cnt, b2):
    mesh = plsc.VectorSubcoreMesh(core_axis_name="c", subcore_axis_name="s")
    upb = b2 // 16
    ushift = upb.bit_length() - 1

    def body(*refs):
        if with_cnt:
            (tab, bsrc, bdst, bcnt, z64, zc, ones_hbm, out_agg, out_cnt,
             src_v, dst_v, rows_v, ones_v, bcnt_v,
             tab_sh, agg_sh, cnt_sh, sem, sem2, sem3) = refs
        else:
            (tab, bsrc, bdst, bcnt, z64, out_agg,
             src_v, dst_v, rows_v, bcnt_v,
             tab_sh, agg_sh, sem, sem2, sem3) = refs
        c = lax.axis_index("c")
        s = lax.axis_index("s")
        wid = s * _NC + c
        iota = lax.iota(jnp.int32, 16)
        rp = 640

        @pl.when(s < _NS - 1)
        def _():
            pltpu.sync_copy(tab.at[pl.ds(s * rp, rp)], tab_sh.at[pl.ds(s * rp, rp)])

        @pl.when(s == _NS - 1)
        def _():
            pltpu.sync_copy(tab.at[pl.ds(9600, 400)], tab_sh.at[pl.ds(9600, 400)])

        if with_cnt:
            pltpu.sync_copy(ones_hbm, ones_v)
        pltpu.sync_copy(bcnt.at[wid], bcnt_v)
        cnt_row = bcnt_v[...]

        zrows = CHUNK // _NS
        for k0 in range(K_CHUNKS):
            pltpu.sync_copy(z64.at[pl.ds(s * zrows, zrows)],
                            agg_sh.at[pl.ds(s * zrows, zrows)])
            if with_cnt:
                pltpu.sync_copy(zc.at[pl.ds(s * zrows, zrows)],
                                cnt_sh.at[pl.ds(s * zrows, zrows)])

            @pl.when(s == 0)
            def _():
                pltpu.sync_copy(z64.at[pl.ds(CHUNK, 8)], agg_sh.at[pl.ds(CHUNK, 8)])
                if with_cnt:
                    pltpu.sync_copy(zc.at[pl.ds(CHUNK, 8)], cnt_sh.at[pl.ds(CHUNK, 8)])

            plsc.subcore_barrier()
            n16 = cnt_row[k0]
            nfull = lax.shift_right_logical(n16, ushift)
            rem = lax.bitwise_and(n16, upb - 1)

            def trip(t, _, k0=k0):
                base = t * b2
                d1 = pltpu.async_copy(bsrc.at[k0, wid, pl.ds(base, b2)], src_v, sem)
                d2 = pltpu.async_copy(bdst.at[k0, wid, pl.ds(base, b2)], dst_v, sem2)
                d1.wait()
                g = pltpu.async_copy(tab_sh.at[src_v], rows_v, sem3)
                d2.wait()
                g.wait()
                s1 = pltpu.async_copy(rows_v, agg_sh.at[dst_v], sem, add=True)
                if with_cnt:
                    pltpu.async_copy(ones_v, cnt_sh.at[dst_v], sem2, add=True).wait()
                s1.wait()
                return _

            lax.fori_loop(0, nfull, trip, 0)

            @pl.when(rem > 0)
            def _(k0=k0, nfull=nfull, rem=rem):
                base = nfull * b2
                pltpu.sync_copy(bsrc.at[k0, wid, pl.ds(base, b2)], src_v)
                pltpu.sync_copy(bdst.at[k0, wid, pl.ds(base, b2)], dst_v)

                def fill(j, _):
                    src_v[pl.ds(j * 16, 16)] = iota
                    dst_v[pl.ds(j * 16, 16)] = jnp.full((16,), _DUMP, jnp.int32)
                    return _

                lax.fori_loop(rem, upb, fill, 0)
                pltpu.async_copy(tab_sh.at[src_v], rows_v, sem3).wait()
                pltpu.sync_copy(rows_v, agg_sh.at[dst_v], add=True)
                if with_cnt:
                    pltpu.sync_copy(ones_v, cnt_sh.at[dst_v], add=True)

            plsc.subcore_barrier()
            pltpu.sync_copy(agg_sh.at[pl.ds(s * zrows, zrows)],
                            out_agg.at[c, pl.ds(k0 * CHUNK + s * zrows, zrows)])
            if with_cnt:
                pltpu.sync_copy(cnt_sh.at[pl.ds(s * zrows, zrows)],
                                out_cnt.at[c, pl.ds(k0 * CHUNK + s * zrows, zrows)])
            plsc.subcore_barrier()

    if with_cnt:
        out_type = (jax.ShapeDtypeStruct((_NC, P_FLOW, D_H), jnp.float32),
                    jax.ShapeDtypeStruct((_NC, P_FLOW, _CW), jnp.float32))
        scratch = [
            pltpu.VMEM((b2,), jnp.int32),
            pltpu.VMEM((b2,), jnp.int32),
            pltpu.VMEM((b2, D_H), jnp.float32),
            pltpu.VMEM((b2, _CW), jnp.float32),
            pltpu.VMEM((16,), jnp.int32),
            pltpu.VMEM_SHARED((N_HOST, D_H), jnp.float32),
            pltpu.VMEM_SHARED((CHUNK + 8, D_H), jnp.float32),
            pltpu.VMEM_SHARED((CHUNK + 8, _CW), jnp.float32),
            pltpu.SemaphoreType.DMA,
            pltpu.SemaphoreType.DMA,
            pltpu.SemaphoreType.DMA,
        ]
    else:
        out_type = jax.ShapeDtypeStruct((_NC, P_FLOW, D_H), jnp.float32)
        scratch = [
            pltpu.VMEM((b2,), jnp.int32),
            pltpu.VMEM((b2,), jnp.int32),
            pltpu.VMEM((b2, D_H), jnp.float32),
            pltpu.VMEM((16,), jnp.int32),
            pltpu.VMEM_SHARED((N_HOST, D_H), jnp.float32),
            pltpu.VMEM_SHARED((CHUNK + 8, D_H), jnp.float32),
            pltpu.SemaphoreType.DMA,
            pltpu.SemaphoreType.DMA,
            pltpu.SemaphoreType.DMA,
        ]
    return pl.kernel(
        body,
        out_type=out_type,
        mesh=mesh,
        scratch_types=scratch,
        compiler_params=pltpu.CompilerParams(use_tc_tiling_on_sc=False,
                                             needs_layout_passes=False),
    )


def _sc_cnt_flow_body(bdst, bcnt, zc, ones_hbm, out_cnt,
                      dst_v, ones_v, bcnt_v, cnt_sh, sem, sem2):
    c = lax.axis_index("c")
    s = lax.axis_index("s")
    wid = s * _NC + c
    iota = lax.iota(jnp.int32, 16)
    b2c = 512
    pltpu.sync_copy(ones_hbm, ones_v)
    pltpu.sync_copy(bcnt.at[wid], bcnt_v)
    cnt_row = bcnt_v[...]
    zrows = CHUNK // _NS
    for k0 in range(K_CHUNKS):
        pltpu.sync_copy(zc.at[pl.ds(s * zrows, zrows)],
                        cnt_sh.at[pl.ds(s * zrows, zrows)])

        @pl.when(s == 0)
        def _():
            pltpu.sync_copy(zc.at[pl.ds(CHUNK, 8)], cnt_sh.at[pl.ds(CHUNK, 8)])

        plsc.subcore_barrier()
        n16 = cnt_row[k0]
        nfull = lax.shift_right_logical(n16, 5)
        rem = lax.bitwise_and(n16, 31)

        def trip(t, _, k0=k0):
            pltpu.sync_copy(bdst.at[k0, wid, pl.ds(t * b2c, b2c)], dst_v)
            pltpu.sync_copy(ones_v, cnt_sh.at[dst_v], add=True)
            return _

        lax.fori_loop(0, nfull, trip, 0)

        @pl.when(rem > 0)
        def _(k0=k0, nfull=nfull, rem=rem):
            pltpu.sync_copy(bdst.at[k0, wid, pl.ds(nfull * b2c, b2c)], dst_v)

            def fill(j, _):
                dst_v[pl.ds(j * 16, 16)] = jnp.full((16,), _DUMP, jnp.int32)
                return _

            lax.fori_loop(rem, b2c // 16, fill, 0)
            pltpu.sync_copy(ones_v, cnt_sh.at[dst_v], add=True)

        plsc.subcore_barrier()
        pltpu.sync_copy(cnt_sh.at[pl.ds(s * zrows, zrows)],
                        out_cnt.at[c, pl.ds(k0 * CHUNK + s * zrows, zrows)])
        plsc.subcore_barrier()


def _sc_cnt_flow(bdst, bcnt):
    mesh = plsc.VectorSubcoreMesh(core_axis_name="c", subcore_axis_name="s")
    zc = jnp.zeros((CHUNK + 8, _CW), jnp.float32)
    ones = jnp.ones((512, _CW), jnp.float32)
    f = pl.kernel(
        _sc_cnt_flow_body,
        out_type=jax.ShapeDtypeStruct((_NC, P_FLOW, _CW), jnp.float32),
        mesh=mesh,
        scratch_types=[
            pltpu.VMEM((512,), jnp.int32),
            pltpu.VMEM((512, _CW), jnp.float32),
            pltpu.VMEM((16,), jnp.int32),
            pltpu.VMEM_SHARED((CHUNK + 8, _CW), jnp.float32),
            pltpu.SemaphoreType.DMA,
            pltpu.SemaphoreType.DMA,
        ],
        compiler_params=pltpu.CompilerParams(use_tc_tiling_on_sc=False,
                                             needs_layout_passes=False),
    )
    return f(bdst, bcnt, zc, ones)


def _sc_agg_flow_l0(h_tab, bsrc, bdst, bcnt):
    z64 = jnp.zeros((CHUNK + 8, D_H), jnp.float32)
    zc = jnp.zeros((CHUNK + 8, _CW), jnp.float32)
    ones = jnp.ones((_B2, _CW), jnp.float32)
    return _make_sc_agg_flow(True, _B2)(h_tab, bsrc, bdst, bcnt, z64, zc, ones)


def _sc_agg_flow_l1(h_tab, bsrc, bdst, bcnt):
    z64 = jnp.zeros((CHUNK + 8, D_H), jnp.float32)
    return _make_sc_agg_flow(False, 256)(h_tab, bsrc, bdst, bcnt, z64)


def _scaffold_agg(x_src, src, dst, n_dst, p_rows):
    m = jnp.take(x_src, src, axis=0)
    s = jax.ops.segment_sum(m, dst, num_segments=n_dst)
    c = jax.ops.segment_sum(jnp.ones((E, _CW), jnp.float32), dst, num_segments=n_dst)
    pad = p_rows - n_dst
    if pad:
        s = jnp.pad(s, ((0, pad), (0, 0)))
        c = jnp.pad(c, ((0, pad), (0, 0)))
    parts = jnp.stack([s, jnp.zeros_like(s)])
    cnt = jnp.stack([c, jnp.zeros_like(c)])
    return parts, cnt


def kernel(x_host, x_flow, ei_h2f_src, ei_h2f_dst, ei_f2h_src, ei_f2h_dst,
           W_host, b_host, W_flow, b_flow,
           Wl_h2f_0, bl_h2f_0, Wr_h2f_0, Wl_f2h_0, bl_f2h_0, Wr_f2h_0,
           Wl_h2f_1, bl_h2f_1, Wr_h2f_1, Wl_f2h_1, bl_f2h_1, Wr_f2h_1,
           W_out, b_out):
    h0 = _tc_proj_relu(x_host, W_host, b_host, rb=2000)
    f0 = _tc_proj_relu(x_flow, W_flow, b_flow, rb=2000)

    bsrc, bdstl, bcnt = _sc_bucket(ei_h2f_src, ei_h2f_dst)
    cntf_p = _sc_cnt_flow(bdstl, bcnt)
    aggf0_p = _sc_agg_flow_l1(h0, bsrc, bdstl, bcnt)
    aggh0_p, cnth_p = _sc_agg_host(f0, ei_f2h_src, ei_f2h_dst)

    f = _tc_layer(aggf0_p, cntf_p, f0, Wl_h2f_0, bl_h2f_0, Wr_h2f_0, rb=2000)
    h = _tc_layer(aggh0_p, cnth_p, h0, Wl_f2h_0, bl_f2h_0, Wr_f2h_0, rb=2000)

    aggf1_p = _sc_agg_flow_l1(h, bsrc, bdstl, bcnt)

    return _tc_final(aggf1_p, cntf_p, f, Wl_h2f_1, bl_h2f_1, Wr_h2f_1,
                     W_out, b_out, rb=2000)


# speedup vs baseline: 1.0065x; 1.0065x over previous
import functools

import jax
import jax.numpy as jnp
from jax import lax
from jax.experimental import pallas as pl
from jax.experimental.pallas import tpu as pltpu
from jax.experimental.pallas import tpu_sc as plsc

N_HOST = 10000
N_FLOW = 100000
E = 600000
D_IN = 128
D_H = 64

CHUNK = 16384
K_CHUNKS = -(-N_FLOW // CHUNK)
P_FLOW = K_CHUNKS * CHUNK
P_HOST = 10240


def _proj_body(x_ref, w_ref, b_ref, o_ref):
    y = jnp.dot(x_ref[...], w_ref[...], preferred_element_type=jnp.float32)
    o_ref[...] = jnp.maximum(y + b_ref[...], 0.0)


def _tc_proj_relu(x, w, b, rb):
    n = x.shape[0]
    return pl.pallas_call(
        _proj_body,
        grid=(n // rb,),
        in_specs=[
            pl.BlockSpec((rb, D_IN), lambda i: (i, 0)),
            pl.BlockSpec((D_IN, D_H), lambda i: (0, 0)),
            pl.BlockSpec((1, D_H), lambda i: (0, 0)),
        ],
        out_specs=pl.BlockSpec((rb, D_H), lambda i: (i, 0)),
        out_shape=jax.ShapeDtypeStruct((n, D_H), jnp.float32),
    )(x, w, b.reshape(1, D_H))


def _layer_body(parts_ref, cnt_ref, prev_ref, wl_ref, bl_ref, wr_ref, o_ref):
    s = parts_ref[0] + parts_ref[1]
    c = jnp.maximum(cnt_ref[0, :, 0:1] + cnt_ref[1, :, 0:1], 1.0)
    agg = s / c
    y = (jnp.dot(agg, wl_ref[...], preferred_element_type=jnp.float32)
         + bl_ref[...]
         + jnp.dot(prev_ref[...], wr_ref[...], preferred_element_type=jnp.float32))
    o_ref[...] = jnp.where(y >= 0.0, y, 0.01 * y)


def _tc_layer(parts, cnt, prev, wl, bl, wr, rb):
    n = prev.shape[0]
    return pl.pallas_call(
        _layer_body,
        grid=(n // rb,),
        in_specs=[
            pl.BlockSpec((2, rb, D_H), lambda i: (0, i, 0)),
            pl.BlockSpec((2, rb, _CW), lambda i: (0, i, 0)),
            pl.BlockSpec((rb, D_H), lambda i: (i, 0)),
            pl.BlockSpec((D_H, D_H), lambda i: (0, 0)),
            pl.BlockSpec((1, D_H), lambda i: (0, 0)),
            pl.BlockSpec((D_H, D_H), lambda i: (0, 0)),
        ],
        out_specs=pl.BlockSpec((rb, D_H), lambda i: (i, 0)),
        out_shape=jax.ShapeDtypeStruct((n, D_H), jnp.float32),
    )(parts, cnt, prev, wl, bl.reshape(1, D_H), wr)


def _final_body(parts_ref, cnt_ref, prev_ref, wl_ref, bl_ref, wr_ref,
                wo_ref, bo_ref, o_ref):
    s = parts_ref[0] + parts_ref[1]
    c = jnp.maximum(cnt_ref[0, :, 0:1] + cnt_ref[1, :, 0:1], 1.0)
    agg = s / c
    y = (jnp.dot(agg, wl_ref[...], preferred_element_type=jnp.float32)
         + bl_ref[...]
         + jnp.dot(prev_ref[...], wr_ref[...], preferred_element_type=jnp.float32))
    f2 = jnp.where(y >= 0.0, y, 0.01 * y)
    o_ref[...] = (jnp.dot(f2, wo_ref[...], preferred_element_type=jnp.float32)
                  + bo_ref[...])


def _tc_final(parts, cnt, prev, wl, bl, wr, wo, bo, rb):
    n = prev.shape[0]
    d_out = wo.shape[1]
    return pl.pallas_call(
        _final_body,
        grid=(n // rb,),
        in_specs=[
            pl.BlockSpec((2, rb, D_H), lambda i: (0, i, 0)),
            pl.BlockSpec((2, rb, _CW), lambda i: (0, i, 0)),
            pl.BlockSpec((rb, D_H), lambda i: (i, 0)),
            pl.BlockSpec((D_H, D_H), lambda i: (0, 0)),
            pl.BlockSpec((1, D_H), lambda i: (0, 0)),
            pl.BlockSpec((D_H, D_H), lambda i: (0, 0)),
            pl.BlockSpec((D_H, d_out), lambda i: (0, 0)),
            pl.BlockSpec((1, d_out), lambda i: (0, 0)),
        ],
        out_specs=pl.BlockSpec((rb, d_out), lambda i: (i, 0)),
        out_shape=jax.ShapeDtypeStruct((n, d_out), jnp.float32),
    )(parts, cnt, prev, wl, bl.reshape(1, D_H), wr, wo, bo.reshape(1, d_out))


_NC, _NS = 2, 16
_NW = _NC * _NS
_CW = 8

_BH = 960
_HSTEPS = E // _BH


def _sc_agg_host_body(tab, src, dst, ones_hbm, z64, zc,
                      out_agg, out_cnt,
                      src_v, dst_v, rows_v, ones_v, agg_sh, cnt_sh,
                      sem, sem2, sem3):
    c = lax.axis_index("c")
    s = lax.axis_index("s")
    wid = s * _NC + c
    rp = P_HOST // _NS
    pltpu.sync_copy(ones_hbm, ones_v)
    pltpu.sync_copy(z64.at[pl.ds(s * rp, rp)], agg_sh.at[pl.ds(s * rp, rp)])
    pltpu.sync_copy(zc.at[pl.ds(s * rp, rp)], cnt_sh.at[pl.ds(s * rp, rp)])
    plsc.subcore_barrier()

    def step(i, carry):
        chunk = i * _NW + wid

        @pl.when(chunk < _HSTEPS)
        def _():
            base = chunk * _BH
            d1 = pltpu.async_copy(src.at[pl.ds(base, _BH)], src_v, sem)
            d2 = pltpu.async_copy(dst.at[pl.ds(base, _BH)], dst_v, sem2)
            d1.wait()
            g = pltpu.async_copy(tab.at[src_v], rows_v, sem3)
            d2.wait()
            g.wait()
            s1 = pltpu.async_copy(rows_v, agg_sh.at[dst_v], sem, add=True)
            pltpu.async_copy(ones_v, cnt_sh.at[dst_v], sem2, add=True).wait()
            s1.wait()

        return carry

    lax.fori_loop(0, (_HSTEPS + _NW - 1) // _NW, step, 0)
    plsc.subcore_barrier()
    pltpu.sync_copy(agg_sh.at[pl.ds(s * rp, rp)],
                    out_agg.at[c, pl.ds(s * rp, rp)])
    pltpu.sync_copy(cnt_sh.at[pl.ds(s * rp, rp)],
                    out_cnt.at[c, pl.ds(s * rp, rp)])


def _sc_agg_host(f0, src, dst):
    mesh = plsc.VectorSubcoreMesh(core_axis_name="c", subcore_axis_name="s")
    ones = jnp.ones((_BH, _CW), jnp.float32)
    z64 = jnp.zeros((P_HOST, D_H), jnp.float32)
    zc = jnp.zeros((P_HOST, _CW), jnp.float32)
    f = pl.kernel(
        _sc_agg_host_body,
        out_type=(jax.ShapeDtypeStruct((_NC, P_HOST, D_H), jnp.float32),
                  jax.ShapeDtypeStruct((_NC, P_HOST, _CW), jnp.float32)),
        mesh=mesh,
        scratch_types=[
            pltpu.VMEM((_BH,), jnp.int32),
            pltpu.VMEM((_BH,), jnp.int32),
            pltpu.VMEM((_BH, D_H), jnp.float32),
            pltpu.VMEM((_BH, _CW), jnp.float32),
            pltpu.VMEM_SHARED((P_HOST, D_H), jnp.float32),
            pltpu.VMEM_SHARED((P_HOST, _CW), jnp.float32),
            pltpu.SemaphoreType.DMA,
            pltpu.SemaphoreType.DMA,
            pltpu.SemaphoreType.DMA,
        ],
        compiler_params=pltpu.CompilerParams(use_tc_tiling_on_sc=False),
    )
    return f(f0, src, dst, ones, z64, zc)


_SHIFT = 14
_MASK = CHUNK - 1
_DUMP = CHUNK
_BE = 960
_NCH = E // _BE
_F = 256
_CAPU = 19456
_B2 = 128


def _sc_bucket_body(src, dst, out_src, out_dst, out_cnt,
                    src_c, dst_c, buf_s, buf_d, cnt_v):
    c = lax.axis_index("c")
    s = lax.axis_index("s")
    wid = s * _NC + c
    iota = lax.iota(jnp.int32, 16)
    base_trips = _NCH // _NW
    trips = jnp.where(wid < _NCH - base_trips * _NW, base_trips + 1, base_trips)

    def chunk_step(i, carry):
        ch = i * _NW + wid
        base = ch * _BE
        pltpu.sync_copy(src.at[pl.ds(base, _BE)], src_c)
        pltpu.sync_copy(dst.at[pl.ds(base, _BE)], dst_c)

        def vreg_step(v, carry2):
            ptrs, fcs = carry2
            s16 = src_c[pl.ds(v * 16, 16)]
            d16 = dst_c[pl.ds(v * 16, 16)]
            k16 = lax.shift_right_logical(d16, _SHIFT)
            dl16 = lax.bitwise_and(d16, _MASK)
            new_ptrs, new_fcs = [], []
            for k0 in range(K_CHUNKS):
                p, fc = ptrs[k0], fcs[k0]
                m = k16 == k0
                n = jnp.sum(m.astype(jnp.int32))
                plsc.store_compressed(buf_s.at[pl.ds(k0 * 512 + p, 16)], s16, mask=m)
                plsc.store_compressed(buf_d.at[pl.ds(k0 * 512 + p, 16)], dl16, mask=m)
                p = p + n
                full = p >= _F

                @pl.when(full)
                def _(k0=k0, fc=fc):
                    pltpu.sync_copy(buf_s.at[pl.ds(k0 * 512, _F)],
                                    out_src.at[k0, wid, pl.ds(fc * _F, _F)])
                    pltpu.sync_copy(buf_d.at[pl.ds(k0 * 512, _F)],
                                    out_dst.at[k0, wid, pl.ds(fc * _F, _F)])
                    buf_s[pl.ds(k0 * 512, 16)] = buf_s[pl.ds(k0 * 512 + _F, 16)]
                    buf_d[pl.ds(k0 * 512, 16)] = buf_d[pl.ds(k0 * 512 + _F, 16)]

                new_ptrs.append(jnp.where(full, p - _F, p))
                new_fcs.append(jnp.where(full, fc + 1, fc))
            return tuple(new_ptrs), tuple(new_fcs)

        return lax.fori_loop(0, _BE // 16, vreg_step, carry)

    zero = jnp.int32(0)
    ptrs, fcs = lax.fori_loop(
        0, trips, chunk_step,
        (tuple(zero for _ in range(K_CHUNKS)),
         tuple(zero for _ in range(K_CHUNKS))))

    cnts = jnp.zeros((16,), jnp.int32)
    for k0 in range(K_CHUNKS):
        p, fc = ptrs[k0], fcs[k0]
        buf_s[pl.ds(k0 * 512 + p, 16)] = iota
        buf_d[pl.ds(k0 * 512 + p, 16)] = jnp.full((16,), _DUMP, jnp.int32)
        n16 = lax.shift_right_logical(p + 15, 4)

        def fl(j, _, k0=k0, fc=fc):
            pltpu.sync_copy(buf_s.at[pl.ds(k0 * 512 + j * 16, 16)],
                            out_src.at[k0, wid, pl.ds(fc * _F + j * 16, 16)])
            pltpu.sync_copy(buf_d.at[pl.ds(k0 * 512 + j * 16, 16)],
                            out_dst.at[k0, wid, pl.ds(fc * _F + j * 16, 16)])
            return _

        lax.fori_loop(0, n16, fl, 0)
        cnts = cnts + jnp.where(iota == k0, fc * (_F // 16) + n16, 0)
    cnt_v[...] = cnts
    pltpu.sync_copy(cnt_v, out_cnt.at[wid])


def _sc_bucket(src, dst):
    mesh = plsc.VectorSubcoreMesh(core_axis_name="c", subcore_axis_name="s")
    f = pl.kernel(
        _sc_bucket_body,
        out_type=(jax.ShapeDtypeStruct((K_CHUNKS, _NW, _CAPU), jnp.int32),
                  jax.ShapeDtypeStruct((K_CHUNKS, _NW, _CAPU), jnp.int32),
                  jax.ShapeDtypeStruct((_NW, 16), jnp.int32)),
        mesh=mesh,
        scratch_types=[
            pltpu.VMEM((_BE,), jnp.int32),
            pltpu.VMEM((_BE,), jnp.int32),
            pltpu.VMEM((K_CHUNKS * 512,), jnp.int32),
            pltpu.VMEM((K_CHUNKS * 512,), jnp.int32),
            pltpu.VMEM((16,), jnp.int32),
        ],
        compiler_params=pltpu.CompilerParams(use_tc_tiling_on_sc=False,
                                             needs_layout_passes=False),
    )
    return f(src, dst)


_B2F = 128


def _make_sc_agg_flow():
    mesh = plsc.VectorSubcoreMesh(core_axis_name="c", subcore_axis_name="s")
    b2 = _B2F
    upb = b2 // 16

    def body(tab, bsrc, bdst, bcnt, z64, out_agg,
             srcA, srcB, dstA, dstB, rowsA, rowsB, bcnt_v,
             tab_sh, agg_sh, sA1, sA2, sB1, sB2, sG1, sG2):
        c = lax.axis_index("c")
        s = lax.axis_index("s")
        wid = s * _NC + c
        iota = lax.iota(jnp.int32, 16)
        rp = 640

        @pl.when(s < _NS - 1)
        def _():
            pltpu.sync_copy(tab.at[pl.ds(s * rp, rp)], tab_sh.at[pl.ds(s * rp, rp)])

        @pl.when(s == _NS - 1)
        def _():
            pltpu.sync_copy(tab.at[pl.ds(9600, 400)], tab_sh.at[pl.ds(9600, 400)])

        pltpu.sync_copy(bcnt.at[wid], bcnt_v)
        cnt_row = bcnt_v[...]

        zrows = CHUNK // _NS
        for k0 in range(K_CHUNKS):
            pltpu.sync_copy(z64.at[pl.ds(s * zrows, zrows)],
                            agg_sh.at[pl.ds(s * zrows, zrows)])

            @pl.when(s == 0)
            def _():
                pltpu.sync_copy(z64.at[pl.ds(CHUNK, 8)], agg_sh.at[pl.ds(CHUNK, 8)])

            plsc.subcore_barrier()
            n16 = cnt_row[k0]
            nfull = lax.shift_right_logical(n16, 3)
            rem = lax.bitwise_and(n16, 7)
            npair = lax.shift_right_logical(nfull, 1)
            odd = lax.bitwise_and(nfull, 1)

            def pair(i, _, k0=k0):
                bA = (2 * i) * b2
                bB = (2 * i + 1) * b2
                dA1 = pltpu.async_copy(bsrc.at[k0, wid, pl.ds(bA, b2)], srcA, sA1)
                dA2 = pltpu.async_copy(bdst.at[k0, wid, pl.ds(bA, b2)], dstA, sA2)
                dA1.wait()
                gA = pltpu.async_copy(tab_sh.at[srcA], rowsA, sG1)
                dB1 = pltpu.async_copy(bsrc.at[k0, wid, pl.ds(bB, b2)], srcB, sB1)
                dB2 = pltpu.async_copy(bdst.at[k0, wid, pl.ds(bB, b2)], dstB, sB2)
                gA.wait()
                dA2.wait()
                scA = pltpu.async_copy(rowsA, agg_sh.at[dstA], sA1, add=True)
                dB1.wait()
                gB = pltpu.async_copy(tab_sh.at[srcB], rowsB, sG2)
                gB.wait()
                dB2.wait()
                scB = pltpu.async_copy(rowsB, agg_sh.at[dstB], sB1, add=True)
                scA.wait()
                scB.wait()
                return _

            lax.fori_loop(0, npair, pair, 0)

            @pl.when(odd == 1)
            def _(k0=k0, npair=npair):
                base = 2 * npair * b2
                d1 = pltpu.async_copy(bsrc.at[k0, wid, pl.ds(base, b2)], srcA, sA1)
                d2 = pltpu.async_copy(bdst.at[k0, wid, pl.ds(base, b2)], dstA, sA2)
                d1.wait()
                g = pltpu.async_copy(tab_sh.at[srcA], rowsA, sG1)
                g.wait()
                d2.wait()
                pltpu.sync_copy(rowsA, agg_sh.at[dstA], add=True)

            @pl.when(rem > 0)
            def _(k0=k0, nfull=nfull, rem=rem):
                base = nfull * b2
                pltpu.sync_copy(bsrc.at[k0, wid, pl.ds(base, b2)], srcA)
                pltpu.sync_copy(bdst.at[k0, wid, pl.ds(base, b2)], dstA)

                def fill(j, _):
                    srcA[pl.ds(j * 16, 16)] = iota
                    dstA[pl.ds(j * 16, 16)] = jnp.full((16,), _DUMP, jnp.int32)
                    return _

                lax.fori_loop(rem, upb, fill, 0)
                pltpu.async_copy(tab_sh.at[srcA], rowsA, sG1).wait()
                pltpu.sync_copy(rowsA, agg_sh.at[dstA], add=True)

            plsc.subcore_barrier()
            pltpu.sync_copy(agg_sh.at[pl.ds(s * zrows, zrows)],
                            out_agg.at[c, pl.ds(k0 * CHUNK + s * zrows, zrows)])
            plsc.subcore_barrier()

    return pl.kernel(
        body,
        out_type=jax.ShapeDtypeStruct((_NC, P_FLOW, D_H), jnp.float32),
        mesh=mesh,
        scratch_types=[
            pltpu.VMEM((b2,), jnp.int32),
            pltpu.VMEM((b2,), jnp.int32),
            pltpu.VMEM((b2,), jnp.int32),
            pltpu.VMEM((b2,), jnp.int32),
            pltpu.VMEM((b2, D_H), jnp.float32),
            pltpu.VMEM((b2, D_H), jnp.float32),
            pltpu.VMEM((16,), jnp.int32),
            pltpu.VMEM_SHARED((N_HOST, D_H), jnp.float32),
            pltpu.VMEM_SHARED((CHUNK + 8, D_H), jnp.float32),
            pltpu.SemaphoreType.DMA,
            pltpu.SemaphoreType.DMA,
            pltpu.SemaphoreType.DMA,
            pltpu.SemaphoreType.DMA,
            pltpu.SemaphoreType.DMA,
            pltpu.SemaphoreType.DMA,
        ],
        compiler_params=pltpu.CompilerParams(use_tc_tiling_on_sc=False,
                                             needs_layout_passes=False),
    )


def _sc_agg_flow(h_tab, bsrc, bdst, bcnt):
    z64 = jnp.zeros((CHUNK + 8, D_H), jnp.float32)
    return _make_sc_agg_flow()(h_tab, bsrc, bdst, bcnt, z64)


def _sc_cnt_flow_body(bdst, bcnt, zc, ones_hbm, out_cnt,
                      dst_v, ones_v, bcnt_v, cnt_sh, sem, sem2):
    c = lax.axis_index("c")
    s = lax.axis_index("s")
    wid = s * _NC + c
    iota = lax.iota(jnp.int32, 16)
    b2c = 512
    pltpu.sync_copy(ones_hbm, ones_v)
    pltpu.sync_copy(bcnt.at[wid], bcnt_v)
    cnt_row = bcnt_v[...]
    zrows = CHUNK // _NS
    for k0 in range(K_CHUNKS):
        pltpu.sync_copy(zc.at[pl.ds(s * zrows, zrows)],
                        cnt_sh.at[pl.ds(s * zrows, zrows)])

        @pl.when(s == 0)
        def _():
            pltpu.sync_copy(zc.at[pl.ds(CHUNK, 8)], cnt_sh.at[pl.ds(CHUNK, 8)])

        plsc.subcore_barrier()
        n16 = cnt_row[k0]
        nfull = lax.shift_right_logical(n16, 5)
        rem = lax.bitwise_and(n16, 31)

        def trip(t, _, k0=k0):
            pltpu.sync_copy(bdst.at[k0, wid, pl.ds(t * b2c, b2c)], dst_v)
            pltpu.sync_copy(ones_v, cnt_sh.at[dst_v], add=True)
            return _

        lax.fori_loop(0, nfull, trip, 0)

        @pl.when(rem > 0)
        def _(k0=k0, nfull=nfull, rem=rem):
            pltpu.sync_copy(bdst.at[k0, wid, pl.ds(nfull * b2c, b2c)], dst_v)

            def fill(j, _):
                dst_v[pl.ds(j * 16, 16)] = jnp.full((16,), _DUMP, jnp.int32)
                return _

            lax.fori_loop(rem, b2c // 16, fill, 0)
            pltpu.sync_copy(ones_v, cnt_sh.at[dst_v], add=True)

        plsc.subcore_barrier()
        pltpu.sync_copy(cnt_sh.at[pl.ds(s * zrows, zrows)],
                        out_cnt.at[c, pl.ds(k0 * CHUNK + s * zrows, zrows)])
        plsc.subcore_barrier()


def _sc_cnt_flow(bdst, bcnt):
    mesh = plsc.VectorSubcoreMesh(core_axis_name="c", subcore_axis_name="s")
    zc = jnp.zeros((CHUNK + 8, _CW), jnp.float32)
    ones = jnp.ones((512, _CW), jnp.float32)
    f = pl.kernel(
        _sc_cnt_flow_body,
        out_type=jax.ShapeDtypeStruct((_NC, P_FLOW, _CW), jnp.float32),
        mesh=mesh,
        scratch_types=[
            pltpu.VMEM((512,), jnp.int32),
            pltpu.VMEM((512, _CW), jnp.float32),
            pltpu.VMEM((16,), jnp.int32),
            pltpu.VMEM_SHARED((CHUNK + 8, _CW), jnp.float32),
            pltpu.SemaphoreType.DMA,
            pltpu.SemaphoreType.DMA,
        ],
        compiler_params=pltpu.CompilerParams(use_tc_tiling_on_sc=False,
                                             needs_layout_passes=False),
    )
    return f(bdst, bcnt, zc, ones)


def _scaffold_agg(x_src, src, dst, n_dst, p_rows):
    m = jnp.take(x_src, src, axis=0)
    s = jax.ops.segment_sum(m, dst, num_segments=n_dst)
    c = jax.ops.segment_sum(jnp.ones((E, _CW), jnp.float32), dst, num_segments=n_dst)
    pad = p_rows - n_dst
    if pad:
        s = jnp.pad(s, ((0, pad), (0, 0)))
        c = jnp.pad(c, ((0, pad), (0, 0)))
    parts = jnp.stack([s, jnp.zeros_like(s)])
    cnt = jnp.stack([c, jnp.zeros_like(c)])
    return parts, cnt


def kernel(x_host, x_flow, ei_h2f_src, ei_h2f_dst, ei_f2h_src, ei_f2h_dst,
           W_host, b_host, W_flow, b_flow,
           Wl_h2f_0, bl_h2f_0, Wr_h2f_0, Wl_f2h_0, bl_f2h_0, Wr_f2h_0,
           Wl_h2f_1, bl_h2f_1, Wr_h2f_1, Wl_f2h_1, bl_f2h_1, Wr_f2h_1,
           W_out, b_out):
    h0 = _tc_proj_relu(x_host, W_host, b_host, rb=2000)
    f0 = _tc_proj_relu(x_flow, W_flow, b_flow, rb=2000)

    bsrc, bdstl, bcnt = _sc_bucket(ei_h2f_src, ei_h2f_dst)
    cntf_p = _sc_cnt_flow(bdstl, bcnt)
    aggf0_p = _sc_agg_flow(h0, bsrc, bdstl, bcnt)
    aggh0_p, cnth_p = _sc_agg_host(f0, ei_f2h_src, ei_f2h_dst)

    f = _tc_layer(aggf0_p, cntf_p, f0, Wl_h2f_0, bl_h2f_0, Wr_h2f_0, rb=2000)
    h = _tc_layer(aggh0_p, cnth_p, h0, Wl_f2h_0, bl_f2h_0, Wr_f2h_0, rb=2000)

    aggf1_p = _sc_agg_flow(h, bsrc, bdstl, bcnt)

    return _tc_final(aggf1_p, cntf_p, f, Wl_h2f_1, bl_h2f_1, Wr_h2f_1,
                     W_out, b_out, rb=2000)


# speedup vs baseline: 1.0268x; 1.0202x over previous
import functools

import jax
import jax.numpy as jnp
from jax import lax
from jax.experimental import pallas as pl
from jax.experimental.pallas import tpu as pltpu
from jax.experimental.pallas import tpu_sc as plsc

N_HOST = 10000
N_FLOW = 100000
E = 600000
D_IN = 128
D_H = 64

CHUNK = 16384
K_CHUNKS = -(-N_FLOW // CHUNK)
P_FLOW = K_CHUNKS * CHUNK
P_HOST = 10240


def _proj_body(x_ref, w_ref, b_ref, o_ref):
    y = jnp.dot(x_ref[...], w_ref[...], preferred_element_type=jnp.float32)
    o_ref[...] = jnp.maximum(y + b_ref[...], 0.0)


def _tc_proj_relu(x, w, b, rb):
    n = x.shape[0]
    return pl.pallas_call(
        _proj_body,
        grid=(n // rb,),
        in_specs=[
            pl.BlockSpec((rb, D_IN), lambda i: (i, 0)),
            pl.BlockSpec((D_IN, D_H), lambda i: (0, 0)),
            pl.BlockSpec((1, D_H), lambda i: (0, 0)),
        ],
        out_specs=pl.BlockSpec((rb, D_H), lambda i: (i, 0)),
        out_shape=jax.ShapeDtypeStruct((n, D_H), jnp.float32),
    )(x, w, b.reshape(1, D_H))


def _layer_body(parts_ref, cnt_ref, prev_ref, wl_ref, bl_ref, wr_ref, o_ref):
    s = parts_ref[0] + parts_ref[1]
    c = jnp.maximum(cnt_ref[0, :, 0:1] + cnt_ref[1, :, 0:1], 1.0)
    agg = s / c
    y = (jnp.dot(agg, wl_ref[...], preferred_element_type=jnp.float32)
         + bl_ref[...]
         + jnp.dot(prev_ref[...], wr_ref[...], preferred_element_type=jnp.float32))
    o_ref[...] = jnp.where(y >= 0.0, y, 0.01 * y)


def _tc_layer(parts, cnt, prev, wl, bl, wr, rb):
    n = prev.shape[0]
    return pl.pallas_call(
        _layer_body,
        grid=(n // rb,),
        in_specs=[
            pl.BlockSpec((2, rb, D_H), lambda i: (0, i, 0)),
            pl.BlockSpec((2, rb, _CW), lambda i: (0, i, 0)),
            pl.BlockSpec((rb, D_H), lambda i: (i, 0)),
            pl.BlockSpec((D_H, D_H), lambda i: (0, 0)),
            pl.BlockSpec((1, D_H), lambda i: (0, 0)),
            pl.BlockSpec((D_H, D_H), lambda i: (0, 0)),
        ],
        out_specs=pl.BlockSpec((rb, D_H), lambda i: (i, 0)),
        out_shape=jax.ShapeDtypeStruct((n, D_H), jnp.float32),
    )(parts, cnt, prev, wl, bl.reshape(1, D_H), wr)


def _final_body(parts_ref, cnt_ref, prev_ref, wl_ref, bl_ref, wr_ref,
                wo_ref, bo_ref, o_ref):
    s = parts_ref[0] + parts_ref[1]
    c = jnp.maximum(cnt_ref[0, :, 0:1] + cnt_ref[1, :, 0:1], 1.0)
    agg = s / c
    y = (jnp.dot(agg, wl_ref[...], preferred_element_type=jnp.float32)
         + bl_ref[...]
         + jnp.dot(prev_ref[...], wr_ref[...], preferred_element_type=jnp.float32))
    f2 = jnp.where(y >= 0.0, y, 0.01 * y)
    o_ref[...] = (jnp.dot(f2, wo_ref[...], preferred_element_type=jnp.float32)
                  + bo_ref[...])


def _tc_final(parts, cnt, prev, wl, bl, wr, wo, bo, rb):
    n = prev.shape[0]
    d_out = wo.shape[1]
    return pl.pallas_call(
        _final_body,
        grid=(n // rb,),
        in_specs=[
            pl.BlockSpec((2, rb, D_H), lambda i: (0, i, 0)),
            pl.BlockSpec((2, rb, _CW), lambda i: (0, i, 0)),
            pl.BlockSpec((rb, D_H), lambda i: (i, 0)),
            pl.BlockSpec((D_H, D_H), lambda i: (0, 0)),
            pl.BlockSpec((1, D_H), lambda i: (0, 0)),
            pl.BlockSpec((D_H, D_H), lambda i: (0, 0)),
            pl.BlockSpec((D_H, d_out), lambda i: (0, 0)),
            pl.BlockSpec((1, d_out), lambda i: (0, 0)),
        ],
        out_specs=pl.BlockSpec((rb, d_out), lambda i: (i, 0)),
        out_shape=jax.ShapeDtypeStruct((n, d_out), jnp.float32),
    )(parts, cnt, prev, wl, bl.reshape(1, D_H), wr, wo, bo.reshape(1, d_out))


_NC, _NS = 2, 16
_NW = _NC * _NS
_CW = 8

_BH = 960
_HSTEPS = E // _BH


def _sc_agg_host_body(tab, src, dst, ones_hbm, z64, zc,
                      out_agg, out_cnt,
                      src_v, dst_v, rows_v, ones_v, agg_sh, cnt_sh,
                      sem, sem2, sem3):
    c = lax.axis_index("c")
    s = lax.axis_index("s")
    wid = s * _NC + c
    rp = P_HOST // _NS
    pltpu.sync_copy(ones_hbm, ones_v)
    pltpu.sync_copy(z64.at[pl.ds(s * rp, rp)], agg_sh.at[pl.ds(s * rp, rp)])
    pltpu.sync_copy(zc.at[pl.ds(s * rp, rp)], cnt_sh.at[pl.ds(s * rp, rp)])
    plsc.subcore_barrier()

    def step(i, carry):
        chunk = i * _NW + wid

        @pl.when(chunk < _HSTEPS)
        def _():
            base = chunk * _BH
            d1 = pltpu.async_copy(src.at[pl.ds(base, _BH)], src_v, sem)
            d2 = pltpu.async_copy(dst.at[pl.ds(base, _BH)], dst_v, sem2)
            d1.wait()
            g = pltpu.async_copy(tab.at[src_v], rows_v, sem3)
            d2.wait()
            g.wait()
            s1 = pltpu.async_copy(rows_v, agg_sh.at[dst_v], sem, add=True)
            pltpu.async_copy(ones_v, cnt_sh.at[dst_v], sem2, add=True).wait()
            s1.wait()

        return carry

    lax.fori_loop(0, (_HSTEPS + _NW - 1) // _NW, step, 0)
    plsc.subcore_barrier()
    pltpu.sync_copy(agg_sh.at[pl.ds(s * rp, rp)],
                    out_agg.at[c, pl.ds(s * rp, rp)])
    pltpu.sync_copy(cnt_sh.at[pl.ds(s * rp, rp)],
                    out_cnt.at[c, pl.ds(s * rp, rp)])


def _sc_agg_host(f0, src, dst):
    mesh = plsc.VectorSubcoreMesh(core_axis_name="c", subcore_axis_name="s")
    ones = jnp.ones((_BH, _CW), jnp.float32)
    z64 = jnp.zeros((P_HOST, D_H), jnp.float32)
    zc = jnp.zeros((P_HOST, _CW), jnp.float32)
    f = pl.kernel(
        _sc_agg_host_body,
        out_type=(jax.ShapeDtypeStruct((_NC, P_HOST, D_H), jnp.float32),
                  jax.ShapeDtypeStruct((_NC, P_HOST, _CW), jnp.float32)),
        mesh=mesh,
        scratch_types=[
            pltpu.VMEM((_BH,), jnp.int32),
            pltpu.VMEM((_BH,), jnp.int32),
            pltpu.VMEM((_BH, D_H), jnp.float32),
            pltpu.VMEM((_BH, _CW), jnp.float32),
            pltpu.VMEM_SHARED((P_HOST, D_H), jnp.float32),
            pltpu.VMEM_SHARED((P_HOST, _CW), jnp.float32),
            pltpu.SemaphoreType.DMA,
            pltpu.SemaphoreType.DMA,
            pltpu.SemaphoreType.DMA,
        ],
        compiler_params=pltpu.CompilerParams(use_tc_tiling_on_sc=False),
    )
    return f(f0, src, dst, ones, z64, zc)


_SHIFT = 14
_MASK = CHUNK - 1
_DUMP = CHUNK
_BE = 960
_NCH = E // _BE
_F = 256
_CAPU = 19456
_B2 = 128


def _sc_bucket_body(src, dst, zc, ones_hbm, out_src, out_dst, out_cnt, out_cntf,
                    src_c, dst_c, buf_s, buf_d, cnt_v, dst_v, ones_v, cnt_sh,
                    isem, isem2):
    c = lax.axis_index("c")
    s = lax.axis_index("s")
    wid = s * _NC + c
    iota = lax.iota(jnp.int32, 16)
    base_trips = _NCH // _NW
    trips = jnp.where(wid < _NCH - base_trips * _NW, base_trips + 1, base_trips)

    def chunk_step(i, carry):
        ch = i * _NW + wid
        base = ch * _BE
        d1 = pltpu.async_copy(src.at[pl.ds(base, _BE)], src_c, isem)
        d2 = pltpu.async_copy(dst.at[pl.ds(base, _BE)], dst_c, isem2)
        d1.wait()
        d2.wait()

        def vreg_step(v, carry2):
            ptrs, fcs = carry2
            s16 = src_c[pl.ds(v * 16, 16)]
            d16 = dst_c[pl.ds(v * 16, 16)]
            k16 = lax.shift_right_logical(d16, _SHIFT)
            dl16 = lax.bitwise_and(d16, _MASK)
            new_ptrs, new_fcs = [], []
            for k0 in range(K_CHUNKS):
                p, fc = ptrs[k0], fcs[k0]
                m = k16 == k0
                n = jnp.sum(m.astype(jnp.int32))
                plsc.store_compressed(buf_s.at[pl.ds(k0 * 512 + p, 16)], s16, mask=m)
                plsc.store_compressed(buf_d.at[pl.ds(k0 * 512 + p, 16)], dl16, mask=m)
                p = p + n
                full = p >= _F

                @pl.when(full)
                def _(k0=k0, fc=fc):
                    pltpu.sync_copy(buf_s.at[pl.ds(k0 * 512, _F)],
                                    out_src.at[k0, wid, pl.ds(fc * _F, _F)])
                    pltpu.sync_copy(buf_d.at[pl.ds(k0 * 512, _F)],
                                    out_dst.at[k0, wid, pl.ds(fc * _F, _F)])
                    buf_s[pl.ds(k0 * 512, 16)] = buf_s[pl.ds(k0 * 512 + _F, 16)]
                    buf_d[pl.ds(k0 * 512, 16)] = buf_d[pl.ds(k0 * 512 + _F, 16)]

                new_ptrs.append(jnp.where(full, p - _F, p))
                new_fcs.append(jnp.where(full, fc + 1, fc))
            return tuple(new_ptrs), tuple(new_fcs)

        return lax.fori_loop(0, _BE // 16, vreg_step, carry)

    zero = jnp.int32(0)
    ptrs, fcs = lax.fori_loop(
        0, trips, chunk_step,
        (tuple(zero for _ in range(K_CHUNKS)),
         tuple(zero for _ in range(K_CHUNKS))))

    cnts = jnp.zeros((16,), jnp.int32)
    for k0 in range(K_CHUNKS):
        p, fc = ptrs[k0], fcs[k0]
        buf_s[pl.ds(k0 * 512 + p, 16)] = iota
        buf_d[pl.ds(k0 * 512 + p, 16)] = jnp.full((16,), _DUMP, jnp.int32)
        n16 = lax.shift_right_logical(p + 15, 4)

        def fl(j, _, k0=k0, fc=fc):
            pltpu.sync_copy(buf_s.at[pl.ds(k0 * 512 + j * 16, 16)],
                            out_src.at[k0, wid, pl.ds(fc * _F + j * 16, 16)])
            pltpu.sync_copy(buf_d.at[pl.ds(k0 * 512 + j * 16, 16)],
                            out_dst.at[k0, wid, pl.ds(fc * _F + j * 16, 16)])
            return _

        lax.fori_loop(0, n16, fl, 0)
        cnts = cnts + jnp.where(iota == k0, fc * (_F // 16) + n16, 0)
    cnt_v[...] = cnts
    pltpu.sync_copy(cnt_v, out_cnt.at[wid])

    cnt_row = cnts
    b2c = 512
    pltpu.sync_copy(ones_hbm, ones_v)
    zrows = CHUNK // _NS
    for k0 in range(K_CHUNKS):
        pltpu.sync_copy(zc.at[pl.ds(s * zrows, zrows)],
                        cnt_sh.at[pl.ds(s * zrows, zrows)])

        @pl.when(s == 0)
        def _():
            pltpu.sync_copy(zc.at[pl.ds(CHUNK, 8)], cnt_sh.at[pl.ds(CHUNK, 8)])

        plsc.subcore_barrier()
        n16c = cnt_row[k0]
        nfullc = lax.shift_right_logical(n16c, 5)
        remc = lax.bitwise_and(n16c, 31)

        def ctrip(t, _, k0=k0):
            pltpu.sync_copy(out_dst.at[k0, wid, pl.ds(t * b2c, b2c)], dst_v)
            pltpu.sync_copy(ones_v, cnt_sh.at[dst_v], add=True)
            return _

        lax.fori_loop(0, nfullc, ctrip, 0)

        @pl.when(remc > 0)
        def _(k0=k0, nfullc=nfullc, remc=remc):
            pltpu.sync_copy(out_dst.at[k0, wid, pl.ds(nfullc * b2c, b2c)], dst_v)

            def fillc(j, _):
                dst_v[pl.ds(j * 16, 16)] = jnp.full((16,), _DUMP, jnp.int32)
                return _

            lax.fori_loop(remc, b2c // 16, fillc, 0)
            pltpu.sync_copy(ones_v, cnt_sh.at[dst_v], add=True)

        plsc.subcore_barrier()
        pltpu.sync_copy(cnt_sh.at[pl.ds(s * zrows, zrows)],
                        out_cntf.at[c, pl.ds(k0 * CHUNK + s * zrows, zrows)])
        plsc.subcore_barrier()


def _sc_bucket(src, dst):
    mesh = plsc.VectorSubcoreMesh(core_axis_name="c", subcore_axis_name="s")
    zc = jnp.zeros((CHUNK + 8, _CW), jnp.float32)
    ones = jnp.ones((512, _CW), jnp.float32)
    f = pl.kernel(
        _sc_bucket_body,
        out_type=(jax.ShapeDtypeStruct((K_CHUNKS, _NW, _CAPU), jnp.int32),
                  jax.ShapeDtypeStruct((K_CHUNKS, _NW, _CAPU), jnp.int32),
                  jax.ShapeDtypeStruct((_NW, 16), jnp.int32),
                  jax.ShapeDtypeStruct((_NC, P_FLOW, _CW), jnp.float32)),
        mesh=mesh,
        scratch_types=[
            pltpu.VMEM((_BE,), jnp.int32),
            pltpu.VMEM((_BE,), jnp.int32),
            pltpu.VMEM((K_CHUNKS * 512,), jnp.int32),
            pltpu.VMEM((K_CHUNKS * 512,), jnp.int32),
            pltpu.VMEM((16,), jnp.int32),
            pltpu.VMEM((512,), jnp.int32),
            pltpu.VMEM((512, _CW), jnp.float32),
            pltpu.VMEM_SHARED((CHUNK + 8, _CW), jnp.float32),
            pltpu.SemaphoreType.DMA,
            pltpu.SemaphoreType.DMA,
        ],
        compiler_params=pltpu.CompilerParams(use_tc_tiling_on_sc=False,
                                             needs_layout_passes=False),
    )
    return f(src, dst, zc, ones)


_B2F = 128


def _make_sc_agg_flow():
    mesh = plsc.VectorSubcoreMesh(core_axis_name="c", subcore_axis_name="s")
    b2 = _B2F
    upb = b2 // 16

    def body(tab, bsrc, bdst, bcnt, z64, out_agg,
             srcA, srcB, dstA, dstB, rowsA, rowsB, bcnt_v,
             tab_sh, agg_sh, sA1, sA2, sB1, sB2, sG1, sG2):
        c = lax.axis_index("c")
        s = lax.axis_index("s")
        wid = s * _NC + c
        iota = lax.iota(jnp.int32, 16)
        rp = 640

        @pl.when(s < _NS - 1)
        def _():
            pltpu.sync_copy(tab.at[pl.ds(s * rp, rp)], tab_sh.at[pl.ds(s * rp, rp)])

        @pl.when(s == _NS - 1)
        def _():
            pltpu.sync_copy(tab.at[pl.ds(9600, 400)], tab_sh.at[pl.ds(9600, 400)])

        pltpu.sync_copy(bcnt.at[wid], bcnt_v)
        cnt_row = bcnt_v[...]

        zrows = CHUNK // _NS
        for k0 in range(K_CHUNKS):
            pltpu.sync_copy(z64.at[pl.ds(s * zrows, zrows)],
                            agg_sh.at[pl.ds(s * zrows, zrows)])

            @pl.when(s == 0)
            def _():
                pltpu.sync_copy(z64.at[pl.ds(CHUNK, 8)], agg_sh.at[pl.ds(CHUNK, 8)])

            plsc.subcore_barrier()
            n16 = cnt_row[k0]
            nfull = lax.shift_right_logical(n16, 3)
            rem = lax.bitwise_and(n16, 7)
            npair = lax.shift_right_logical(nfull, 1)
            odd = lax.bitwise_and(nfull, 1)

            def pair(i, _, k0=k0):
                bA = (2 * i) * b2
                bB = (2 * i + 1) * b2
                dA1 = pltpu.async_copy(bsrc.at[k0, wid, pl.ds(bA, b2)], srcA, sA1)
                dA2 = pltpu.async_copy(bdst.at[k0, wid, pl.ds(bA, b2)], dstA, sA2)
                dA1.wait()
                gA = pltpu.async_copy(tab_sh.at[srcA], rowsA, sG1)
                dB1 = pltpu.async_copy(bsrc.at[k0, wid, pl.ds(bB, b2)], srcB, sB1)
                dB2 = pltpu.async_copy(bdst.at[k0, wid, pl.ds(bB, b2)], dstB, sB2)
                gA.wait()
                dA2.wait()
                scA = pltpu.async_copy(rowsA, agg_sh.at[dstA], sA1, add=True)
                dB1.wait()
                gB = pltpu.async_copy(tab_sh.at[srcB], rowsB, sG2)
                gB.wait()
                dB2.wait()
                scB = pltpu.async_copy(rowsB, agg_sh.at[dstB], sB1, add=True)
                scA.wait()
                scB.wait()
                return _

            lax.fori_loop(0, npair, pair, 0)

            @pl.when(odd == 1)
            def _(k0=k0, npair=npair):
                base = 2 * npair * b2
                d1 = pltpu.async_copy(bsrc.at[k0, wid, pl.ds(base, b2)], srcA, sA1)
                d2 = pltpu.async_copy(bdst.at[k0, wid, pl.ds(base, b2)], dstA, sA2)
                d1.wait()
                g = pltpu.async_copy(tab_sh.at[srcA], rowsA, sG1)
                g.wait()
                d2.wait()
                pltpu.sync_copy(rowsA, agg_sh.at[dstA], add=True)

            @pl.when(rem > 0)
            def _(k0=k0, nfull=nfull, rem=rem):
                base = nfull * b2
                pltpu.sync_copy(bsrc.at[k0, wid, pl.ds(base, b2)], srcA)
                pltpu.sync_copy(bdst.at[k0, wid, pl.ds(base, b2)], dstA)

                def fill(j, _):
                    srcA[pl.ds(j * 16, 16)] = iota
                    dstA[pl.ds(j * 16, 16)] = jnp.full((16,), _DUMP, jnp.int32)
                    return _

                lax.fori_loop(rem, upb, fill, 0)
                pltpu.async_copy(tab_sh.at[srcA], rowsA, sG1).wait()
                pltpu.sync_copy(rowsA, agg_sh.at[dstA], add=True)

            plsc.subcore_barrier()
            pltpu.sync_copy(agg_sh.at[pl.ds(s * zrows, zrows)],
                            out_agg.at[c, pl.ds(k0 * CHUNK + s * zrows, zrows)])
            plsc.subcore_barrier()

    return pl.kernel(
        body,
        out_type=jax.ShapeDtypeStruct((_NC, P_FLOW, D_H), jnp.float32),
        mesh=mesh,
        scratch_types=[
            pltpu.VMEM((b2,), jnp.int32),
            pltpu.VMEM((b2,), jnp.int32),
            pltpu.VMEM((b2,), jnp.int32),
            pltpu.VMEM((b2,), jnp.int32),
            pltpu.VMEM((b2, D_H), jnp.float32),
            pltpu.VMEM((b2, D_H), jnp.float32),
            pltpu.VMEM((16,), jnp.int32),
            pltpu.VMEM_SHARED((N_HOST, D_H), jnp.float32),
            pltpu.VMEM_SHARED((CHUNK + 8, D_H), jnp.float32),
            pltpu.SemaphoreType.DMA,
            pltpu.SemaphoreType.DMA,
            pltpu.SemaphoreType.DMA,
            pltpu.SemaphoreType.DMA,
            pltpu.SemaphoreType.DMA,
            pltpu.SemaphoreType.DMA,
        ],
        compiler_params=pltpu.CompilerParams(use_tc_tiling_on_sc=False,
                                             needs_layout_passes=False),
    )


def _sc_agg_flow(h_tab, bsrc, bdst, bcnt):
    z64 = jnp.zeros((CHUNK + 8, D_H), jnp.float32)
    return _make_sc_agg_flow()(h_tab, bsrc, bdst, bcnt, z64)


def _sc_agg_flow(h_tab, bsrc, bdst, bcnt):
    z64 = jnp.zeros((CHUNK + 8, D_H), jnp.float32)
    return _make_sc_agg_flow()(h_tab, bsrc, bdst, bcnt, z64)


def _sc_cnt_flow_body(bdst, bcnt, zc, ones_hbm, out_cnt,
                      dst_v, ones_v, bcnt_v, cnt_sh, sem, sem2):
    c = lax.axis_index("c")
    s = lax.axis_index("s")
    wid = s * _NC + c
    iota = lax.iota(jnp.int32, 16)
    b2c = 512
    pltpu.sync_copy(ones_hbm, ones_v)
    pltpu.sync_copy(bcnt.at[wid], bcnt_v)
    cnt_row = bcnt_v[...]
    zrows = CHUNK // _NS
    for k0 in range(K_CHUNKS):
        pltpu.sync_copy(zc.at[pl.ds(s * zrows, zrows)],
                        cnt_sh.at[pl.ds(s * zrows, zrows)])

        @pl.when(s == 0)
        def _():
            pltpu.sync_copy(zc.at[pl.ds(CHUNK, 8)], cnt_sh.at[pl.ds(CHUNK, 8)])

        plsc.subcore_barrier()
        n16 = cnt_row[k0]
        nfull = lax.shift_right_logical(n16, 5)
        rem = lax.bitwise_and(n16, 31)

        def trip(t, _, k0=k0):
            pltpu.sync_copy(bdst.at[k0, wid, pl.ds(t * b2c, b2c)], dst_v)
            pltpu.sync_copy(ones_v, cnt_sh.at[dst_v], add=True)
            return _

        lax.fori_loop(0, nfull, trip, 0)

        @pl.when(rem > 0)
        def _(k0=k0, nfull=nfull, rem=rem):
            pltpu.sync_copy(bdst.at[k0, wid, pl.ds(nfull * b2c, b2c)], dst_v)

            def fill(j, _):
                dst_v[pl.ds(j * 16, 16)] = jnp.full((16,), _DUMP, jnp.int32)
                return _

            lax.fori_loop(rem, b2c // 16, fill, 0)
            pltpu.sync_copy(ones_v, cnt_sh.at[dst_v], add=True)

        plsc.subcore_barrier()
        pltpu.sync_copy(cnt_sh.at[pl.ds(s * zrows, zrows)],
                        out_cnt.at[c, pl.ds(k0 * CHUNK + s * zrows, zrows)])
        plsc.subcore_barrier()


def _sc_cnt_flow(bdst, bcnt):
    mesh = plsc.VectorSubcoreMesh(core_axis_name="c", subcore_axis_name="s")
    zc = jnp.zeros((CHUNK + 8, _CW), jnp.float32)
    ones = jnp.ones((512, _CW), jnp.float32)
    f = pl.kernel(
        _sc_cnt_flow_body,
        out_type=jax.ShapeDtypeStruct((_NC, P_FLOW, _CW), jnp.float32),
        mesh=mesh,
        scratch_types=[
            pltpu.VMEM((512,), jnp.int32),
            pltpu.VMEM((512, _CW), jnp.float32),
            pltpu.VMEM((16,), jnp.int32),
            pltpu.VMEM_SHARED((CHUNK + 8, _CW), jnp.float32),
            pltpu.SemaphoreType.DMA,
            pltpu.SemaphoreType.DMA,
        ],
        compiler_params=pltpu.CompilerParams(use_tc_tiling_on_sc=False,
                                             needs_layout_passes=False),
    )
    return f(bdst, bcnt, zc, ones)


def _scaffold_agg(x_src, src, dst, n_dst, p_rows):
    m = jnp.take(x_src, src, axis=0)
    s = jax.ops.segment_sum(m, dst, num_segments=n_dst)
    c = jax.ops.segment_sum(jnp.ones((E, _CW), jnp.float32), dst, num_segments=n_dst)
    pad = p_rows - n_dst
    if pad:
        s = jnp.pad(s, ((0, pad), (0, 0)))
        c = jnp.pad(c, ((0, pad), (0, 0)))
    parts = jnp.stack([s, jnp.zeros_like(s)])
    cnt = jnp.stack([c, jnp.zeros_like(c)])
    return parts, cnt


def kernel(x_host, x_flow, ei_h2f_src, ei_h2f_dst, ei_f2h_src, ei_f2h_dst,
           W_host, b_host, W_flow, b_flow,
           Wl_h2f_0, bl_h2f_0, Wr_h2f_0, Wl_f2h_0, bl_f2h_0, Wr_f2h_0,
           Wl_h2f_1, bl_h2f_1, Wr_h2f_1, Wl_f2h_1, bl_f2h_1, Wr_f2h_1,
           W_out, b_out):
    h0 = _tc_proj_relu(x_host, W_host, b_host, rb=2000)
    f0 = _tc_proj_relu(x_flow, W_flow, b_flow, rb=2000)

    bsrc, bdstl, bcnt, cntf_p = _sc_bucket(ei_h2f_src, ei_h2f_dst)
    aggf0_p = _sc_agg_flow(h0, bsrc, bdstl, bcnt)
    aggh0_p, cnth_p = _sc_agg_host(f0, ei_f2h_src, ei_f2h_dst)

    f = _tc_layer(aggf0_p, cntf_p, f0, Wl_h2f_0, bl_h2f_0, Wr_h2f_0, rb=2000)
    h = _tc_layer(aggh0_p, cnth_p, h0, Wl_f2h_0, bl_f2h_0, Wr_f2h_0, rb=2000)

    aggf1_p = _sc_agg_flow(h, bsrc, bdstl, bcnt)

    return _tc_final(aggf1_p, cntf_p, f, Wl_h2f_1, bl_h2f_1, Wr_h2f_1,
                     W_out, b_out, rb=2000)


# speedup vs baseline: 1.0275x; 1.0007x over previous
import jax
import jax.numpy as jnp
from jax import lax
from jax.experimental import pallas as pl
from jax.experimental.pallas import tpu as pltpu
from jax.experimental.pallas import tpu_sc as plsc

N_HOST = 10000
N_FLOW = 100000
E = 600000
D_IN = 128
D_H = 64

CHUNK = 16384
K_CHUNKS = -(-N_FLOW // CHUNK)
P_FLOW = K_CHUNKS * CHUNK
P_HOST = 10240


def _proj_body(x_ref, w_ref, b_ref, o_ref):
    y = jnp.dot(x_ref[...], w_ref[...], preferred_element_type=jnp.float32)
    o_ref[...] = jnp.maximum(y + b_ref[...], 0.0)


def _tc_proj_relu(x, w, b, rb):
    n = x.shape[0]
    return pl.pallas_call(
        _proj_body,
        grid=(n // rb,),
        in_specs=[
            pl.BlockSpec((rb, D_IN), lambda i: (i, 0)),
            pl.BlockSpec((D_IN, D_H), lambda i: (0, 0)),
            pl.BlockSpec((1, D_H), lambda i: (0, 0)),
        ],
        out_specs=pl.BlockSpec((rb, D_H), lambda i: (i, 0)),
        out_shape=jax.ShapeDtypeStruct((n, D_H), jnp.float32),
    )(x, w, b.reshape(1, D_H))


def _layer_body(parts_ref, cnt_ref, prev_ref, wl_ref, bl_ref, wr_ref, o_ref):
    s = parts_ref[0] + parts_ref[1]
    c = jnp.maximum(cnt_ref[0, :, 0:1] + cnt_ref[1, :, 0:1], 1.0)
    agg = s / c
    y = (jnp.dot(agg, wl_ref[...], preferred_element_type=jnp.float32)
         + bl_ref[...]
         + jnp.dot(prev_ref[...], wr_ref[...], preferred_element_type=jnp.float32))
    o_ref[...] = jnp.where(y >= 0.0, y, 0.01 * y)


def _tc_layer(parts, cnt, prev, wl, bl, wr, rb):
    n = prev.shape[0]
    return pl.pallas_call(
        _layer_body,
        grid=(n // rb,),
        in_specs=[
            pl.BlockSpec((2, rb, D_H), lambda i: (0, i, 0)),
            pl.BlockSpec((2, rb, _CW), lambda i: (0, i, 0)),
            pl.BlockSpec((rb, D_H), lambda i: (i, 0)),
            pl.BlockSpec((D_H, D_H), lambda i: (0, 0)),
            pl.BlockSpec((1, D_H), lambda i: (0, 0)),
            pl.BlockSpec((D_H, D_H), lambda i: (0, 0)),
        ],
        out_specs=pl.BlockSpec((rb, D_H), lambda i: (i, 0)),
        out_shape=jax.ShapeDtypeStruct((n, D_H), jnp.float32),
    )(parts, cnt, prev, wl, bl.reshape(1, D_H), wr)


def _final_body(parts_ref, cnt_ref, prev_ref, wl_ref, bl_ref, wr_ref,
                wo_ref, bo_ref, o_ref):
    s = parts_ref[0] + parts_ref[1]
    c = jnp.maximum(cnt_ref[0, :, 0:1] + cnt_ref[1, :, 0:1], 1.0)
    agg = s / c
    y = (jnp.dot(agg, wl_ref[...], preferred_element_type=jnp.float32)
         + bl_ref[...]
         + jnp.dot(prev_ref[...], wr_ref[...], preferred_element_type=jnp.float32))
    f2 = jnp.where(y >= 0.0, y, 0.01 * y)
    o_ref[...] = (jnp.dot(f2, wo_ref[...], preferred_element_type=jnp.float32)
                  + bo_ref[...])


def _tc_final(parts, cnt, prev, wl, bl, wr, wo, bo, rb):
    n = prev.shape[0]
    d_out = wo.shape[1]
    return pl.pallas_call(
        _final_body,
        grid=(n // rb,),
        in_specs=[
            pl.BlockSpec((2, rb, D_H), lambda i: (0, i, 0)),
            pl.BlockSpec((2, rb, _CW), lambda i: (0, i, 0)),
            pl.BlockSpec((rb, D_H), lambda i: (i, 0)),
            pl.BlockSpec((D_H, D_H), lambda i: (0, 0)),
            pl.BlockSpec((1, D_H), lambda i: (0, 0)),
            pl.BlockSpec((D_H, D_H), lambda i: (0, 0)),
            pl.BlockSpec((D_H, d_out), lambda i: (0, 0)),
            pl.BlockSpec((1, d_out), lambda i: (0, 0)),
        ],
        out_specs=pl.BlockSpec((rb, d_out), lambda i: (i, 0)),
        out_shape=jax.ShapeDtypeStruct((n, d_out), jnp.float32),
    )(parts, cnt, prev, wl, bl.reshape(1, D_H), wr, wo, bo.reshape(1, d_out))


_NC, _NS = 2, 16
_NW = _NC * _NS
_CW = 8

_BH = 960
_HSTEPS = E // _BH


def _sc_agg_host_body(tab, src, dst, ones_hbm, z64, zc,
                      out_agg, out_cnt,
                      src_v, dst_v, rows_v, ones_v, agg_sh, cnt_sh,
                      sem, sem2, sem3):
    c = lax.axis_index("c")
    s = lax.axis_index("s")
    wid = s * _NC + c
    rp = P_HOST // _NS
    pltpu.sync_copy(ones_hbm, ones_v)
    pltpu.sync_copy(z64.at[pl.ds(s * rp, rp)], agg_sh.at[pl.ds(s * rp, rp)])
    pltpu.sync_copy(zc.at[pl.ds(s * rp, rp)], cnt_sh.at[pl.ds(s * rp, rp)])
    plsc.subcore_barrier()

    def step(i, carry):
        chunk = i * _NW + wid

        @pl.when(chunk < _HSTEPS)
        def _():
            base = chunk * _BH
            d1 = pltpu.async_copy(src.at[pl.ds(base, _BH)], src_v, sem)
            d2 = pltpu.async_copy(dst.at[pl.ds(base, _BH)], dst_v, sem2)
            d1.wait()
            g = pltpu.async_copy(tab.at[src_v], rows_v, sem3)
            d2.wait()
            g.wait()
            s1 = pltpu.async_copy(rows_v, agg_sh.at[dst_v], sem, add=True)
            pltpu.async_copy(ones_v, cnt_sh.at[dst_v], sem2, add=True).wait()
            s1.wait()

        return carry

    lax.fori_loop(0, (_HSTEPS + _NW - 1) // _NW, step, 0)
    plsc.subcore_barrier()
    pltpu.sync_copy(agg_sh.at[pl.ds(s * rp, rp)],
                    out_agg.at[c, pl.ds(s * rp, rp)])
    pltpu.sync_copy(cnt_sh.at[pl.ds(s * rp, rp)],
                    out_cnt.at[c, pl.ds(s * rp, rp)])


def _sc_agg_host(f0, src, dst):
    mesh = plsc.VectorSubcoreMesh(core_axis_name="c", subcore_axis_name="s")
    ones = jnp.ones((_BH, _CW), jnp.float32)
    z64 = jnp.zeros((P_HOST, D_H), jnp.float32)
    zc = jnp.zeros((P_HOST, _CW), jnp.float32)
    f = pl.kernel(
        _sc_agg_host_body,
        out_type=(jax.ShapeDtypeStruct((_NC, P_HOST, D_H), jnp.float32),
                  jax.ShapeDtypeStruct((_NC, P_HOST, _CW), jnp.float32)),
        mesh=mesh,
        scratch_types=[
            pltpu.VMEM((_BH,), jnp.int32),
            pltpu.VMEM((_BH,), jnp.int32),
            pltpu.VMEM((_BH, D_H), jnp.float32),
            pltpu.VMEM((_BH, _CW), jnp.float32),
            pltpu.VMEM_SHARED((P_HOST, D_H), jnp.float32),
            pltpu.VMEM_SHARED((P_HOST, _CW), jnp.float32),
            pltpu.SemaphoreType.DMA,
            pltpu.SemaphoreType.DMA,
            pltpu.SemaphoreType.DMA,
        ],
        compiler_params=pltpu.CompilerParams(use_tc_tiling_on_sc=False),
    )
    return f(f0, src, dst, ones, z64, zc)


_SHIFT = 14
_MASK = CHUNK - 1
_DUMP = CHUNK
_BE = 960
_NCH = E // _BE
_F = 256
_CAPU = 19456
_B2 = 128


def _sc_bucket_body(src, dst, zc, ones_hbm, out_src, out_dst, out_cnt, out_cntf,
                    src_c, dst_c, buf_s, buf_d, cnt_v, dst_v, ones_v, cnt_sh,
                    isem, isem2):
    c = lax.axis_index("c")
    s = lax.axis_index("s")
    wid = s * _NC + c
    iota = lax.iota(jnp.int32, 16)
    base_trips = _NCH // _NW
    trips = jnp.where(wid < _NCH - base_trips * _NW, base_trips + 1, base_trips)

    def chunk_step(i, carry):
        ch = i * _NW + wid
        base = ch * _BE
        d1 = pltpu.async_copy(src.at[pl.ds(base, _BE)], src_c, isem)
        d2 = pltpu.async_copy(dst.at[pl.ds(base, _BE)], dst_c, isem2)
        d1.wait()
        d2.wait()

        def vreg_step(v, carry2):
            ptrs, fcs = carry2
            s16 = src_c[pl.ds(v * 16, 16)]
            d16 = dst_c[pl.ds(v * 16, 16)]
            k16 = lax.shift_right_logical(d16, _SHIFT)
            dl16 = lax.bitwise_and(d16, _MASK)
            new_ptrs, new_fcs = [], []
            for k0 in range(K_CHUNKS):
                p, fc = ptrs[k0], fcs[k0]
                m = k16 == k0
                n = jnp.sum(m.astype(jnp.int32))
                plsc.store_compressed(buf_s.at[pl.ds(k0 * 512 + p, 16)], s16, mask=m)
                plsc.store_compressed(buf_d.at[pl.ds(k0 * 512 + p, 16)], dl16, mask=m)
                p = p + n
                full = p >= _F

                @pl.when(full)
                def _(k0=k0, fc=fc):
                    pltpu.sync_copy(buf_s.at[pl.ds(k0 * 512, _F)],
                                    out_src.at[k0, wid, pl.ds(fc * _F, _F)])
                    pltpu.sync_copy(buf_d.at[pl.ds(k0 * 512, _F)],
                                    out_dst.at[k0, wid, pl.ds(fc * _F, _F)])
                    buf_s[pl.ds(k0 * 512, 16)] = buf_s[pl.ds(k0 * 512 + _F, 16)]
                    buf_d[pl.ds(k0 * 512, 16)] = buf_d[pl.ds(k0 * 512 + _F, 16)]

                new_ptrs.append(jnp.where(full, p - _F, p))
                new_fcs.append(jnp.where(full, fc + 1, fc))
            return tuple(new_ptrs), tuple(new_fcs)

        return lax.fori_loop(0, _BE // 16, vreg_step, carry)

    zero = jnp.int32(0)
    ptrs, fcs = lax.fori_loop(
        0, trips, chunk_step,
        (tuple(zero for _ in range(K_CHUNKS)),
         tuple(zero for _ in range(K_CHUNKS))))

    cnts = jnp.zeros((16,), jnp.int32)
    for k0 in range(K_CHUNKS):
        p, fc = ptrs[k0], fcs[k0]
        buf_s[pl.ds(k0 * 512 + p, 16)] = iota
        buf_d[pl.ds(k0 * 512 + p, 16)] = jnp.full((16,), _DUMP, jnp.int32)
        n16 = lax.shift_right_logical(p + 15, 4)

        def fl(j, _, k0=k0, fc=fc):
            pltpu.sync_copy(buf_s.at[pl.ds(k0 * 512 + j * 16, 16)],
                            out_src.at[k0, wid, pl.ds(fc * _F + j * 16, 16)])
            pltpu.sync_copy(buf_d.at[pl.ds(k0 * 512 + j * 16, 16)],
                            out_dst.at[k0, wid, pl.ds(fc * _F + j * 16, 16)])
            return _

        lax.fori_loop(0, n16, fl, 0)
        cnts = cnts + jnp.where(iota == k0, fc * (_F // 16) + n16, 0)
    cnt_v[...] = cnts
    pltpu.sync_copy(cnt_v, out_cnt.at[wid])

    cnt_row = cnts
    b2c = 512
    pltpu.sync_copy(ones_hbm, ones_v)
    zrows = CHUNK // _NS
    for k0 in range(K_CHUNKS):
        pltpu.sync_copy(zc.at[pl.ds(s * zrows, zrows)],
                        cnt_sh.at[pl.ds(s * zrows, zrows)])

        @pl.when(s == 0)
        def _():
            pltpu.sync_copy(zc.at[pl.ds(CHUNK, 8)], cnt_sh.at[pl.ds(CHUNK, 8)])

        plsc.subcore_barrier()
        n16c = cnt_row[k0]
        nfullc = lax.shift_right_logical(n16c, 5)
        remc = lax.bitwise_and(n16c, 31)

        def ctrip(t, _, k0=k0):
            pltpu.sync_copy(out_dst.at[k0, wid, pl.ds(t * b2c, b2c)], dst_v)
            pltpu.sync_copy(ones_v, cnt_sh.at[dst_v], add=True)
            return _

        lax.fori_loop(0, nfullc, ctrip, 0)

        @pl.when(remc > 0)
        def _(k0=k0, nfullc=nfullc, remc=remc):
            pltpu.sync_copy(out_dst.at[k0, wid, pl.ds(nfullc * b2c, b2c)], dst_v)

            def fillc(j, _):
                dst_v[pl.ds(j * 16, 16)] = jnp.full((16,), _DUMP, jnp.int32)
                return _

            lax.fori_loop(remc, b2c // 16, fillc, 0)
            pltpu.sync_copy(ones_v, cnt_sh.at[dst_v], add=True)

        plsc.subcore_barrier()
        pltpu.sync_copy(cnt_sh.at[pl.ds(s * zrows, zrows)],
                        out_cntf.at[c, pl.ds(k0 * CHUNK + s * zrows, zrows)])
        plsc.subcore_barrier()


def _sc_bucket(src, dst):
    mesh = plsc.VectorSubcoreMesh(core_axis_name="c", subcore_axis_name="s")
    zc = jnp.zeros((CHUNK + 8, _CW), jnp.float32)
    ones = jnp.ones((512, _CW), jnp.float32)
    f = pl.kernel(
        _sc_bucket_body,
        out_type=(jax.ShapeDtypeStruct((K_CHUNKS, _NW, _CAPU), jnp.int32),
                  jax.ShapeDtypeStruct((K_CHUNKS, _NW, _CAPU), jnp.int32),
                  jax.ShapeDtypeStruct((_NW, 16), jnp.int32),
                  jax.ShapeDtypeStruct((_NC, P_FLOW, _CW), jnp.float32)),
        mesh=mesh,
        scratch_types=[
            pltpu.VMEM((_BE,), jnp.int32),
            pltpu.VMEM((_BE,), jnp.int32),
            pltpu.VMEM((K_CHUNKS * 512,), jnp.int32),
            pltpu.VMEM((K_CHUNKS * 512,), jnp.int32),
            pltpu.VMEM((16,), jnp.int32),
            pltpu.VMEM((512,), jnp.int32),
            pltpu.VMEM((512, _CW), jnp.float32),
            pltpu.VMEM_SHARED((CHUNK + 8, _CW), jnp.float32),
            pltpu.SemaphoreType.DMA,
            pltpu.SemaphoreType.DMA,
        ],
        compiler_params=pltpu.CompilerParams(use_tc_tiling_on_sc=False,
                                             needs_layout_passes=False),
    )
    return f(src, dst, zc, ones)


_B2F = 128


def _make_sc_agg_flow():
    mesh = plsc.VectorSubcoreMesh(core_axis_name="c", subcore_axis_name="s")
    b2 = _B2F
    upb = b2 // 16

    def body(tab, bsrc, bdst, bcnt, z64, out_agg,
             srcA, srcB, dstA, dstB, rowsA, rowsB, bcnt_v,
             tab_sh, agg_sh, sA1, sA2, sB1, sB2, sG1, sG2):
        c = lax.axis_index("c")
        s = lax.axis_index("s")
        wid = s * _NC + c
        iota = lax.iota(jnp.int32, 16)
        rp = 640

        @pl.when(s < _NS - 1)
        def _():
            pltpu.sync_copy(tab.at[pl.ds(s * rp, rp)], tab_sh.at[pl.ds(s * rp, rp)])

        @pl.when(s == _NS - 1)
        def _():
            pltpu.sync_copy(tab.at[pl.ds(9600, 400)], tab_sh.at[pl.ds(9600, 400)])

        pltpu.sync_copy(bcnt.at[wid], bcnt_v)
        cnt_row = bcnt_v[...]

        zrows = CHUNK // _NS
        for k0 in range(K_CHUNKS):
            pltpu.sync_copy(z64.at[pl.ds(s * zrows, zrows)],
                            agg_sh.at[pl.ds(s * zrows, zrows)])

            @pl.when(s == 0)
            def _():
                pltpu.sync_copy(z64.at[pl.ds(CHUNK, 8)], agg_sh.at[pl.ds(CHUNK, 8)])

            plsc.subcore_barrier()
            n16 = cnt_row[k0]
            nfull = lax.shift_right_logical(n16, 3)
            rem = lax.bitwise_and(n16, 7)
            npair = lax.shift_right_logical(nfull, 1)
            odd = lax.bitwise_and(nfull, 1)

            def pair(i, _, k0=k0):
                bA = (2 * i) * b2
                bB = (2 * i + 1) * b2
                dA1 = pltpu.async_copy(bsrc.at[k0, wid, pl.ds(bA, b2)], srcA, sA1)
                dA2 = pltpu.async_copy(bdst.at[k0, wid, pl.ds(bA, b2)], dstA, sA2)
                dA1.wait()
                gA = pltpu.async_copy(tab_sh.at[srcA], rowsA, sG1)
                dB1 = pltpu.async_copy(bsrc.at[k0, wid, pl.ds(bB, b2)], srcB, sB1)
                dB2 = pltpu.async_copy(bdst.at[k0, wid, pl.ds(bB, b2)], dstB, sB2)
                gA.wait()
                dA2.wait()
                scA = pltpu.async_copy(rowsA, agg_sh.at[dstA], sA1, add=True)
                dB1.wait()
                gB = pltpu.async_copy(tab_sh.at[srcB], rowsB, sG2)
                gB.wait()
                dB2.wait()
                scB = pltpu.async_copy(rowsB, agg_sh.at[dstB], sB1, add=True)
                scA.wait()
                scB.wait()
                return _

            lax.fori_loop(0, npair, pair, 0)

            @pl.when(odd == 1)
            def _(k0=k0, npair=npair):
                base = 2 * npair * b2
                d1 = pltpu.async_copy(bsrc.at[k0, wid, pl.ds(base, b2)], srcA, sA1)
                d2 = pltpu.async_copy(bdst.at[k0, wid, pl.ds(base, b2)], dstA, sA2)
                d1.wait()
                g = pltpu.async_copy(tab_sh.at[srcA], rowsA, sG1)
                g.wait()
                d2.wait()
                pltpu.sync_copy(rowsA, agg_sh.at[dstA], add=True)

            @pl.when(rem > 0)
            def _(k0=k0, nfull=nfull, rem=rem):
                base = nfull * b2
                pltpu.sync_copy(bsrc.at[k0, wid, pl.ds(base, b2)], srcA)
                pltpu.sync_copy(bdst.at[k0, wid, pl.ds(base, b2)], dstA)

                def fill(j, _):
                    srcA[pl.ds(j * 16, 16)] = iota
                    dstA[pl.ds(j * 16, 16)] = jnp.full((16,), _DUMP, jnp.int32)
                    return _

                lax.fori_loop(rem, upb, fill, 0)
                pltpu.async_copy(tab_sh.at[srcA], rowsA, sG1).wait()
                pltpu.sync_copy(rowsA, agg_sh.at[dstA], add=True)

            plsc.subcore_barrier()
            pltpu.sync_copy(agg_sh.at[pl.ds(s * zrows, zrows)],
                            out_agg.at[c, pl.ds(k0 * CHUNK + s * zrows, zrows)])
            plsc.subcore_barrier()

    return pl.kernel(
        body,
        out_type=jax.ShapeDtypeStruct((_NC, P_FLOW, D_H), jnp.float32),
        mesh=mesh,
        scratch_types=[
            pltpu.VMEM((b2,), jnp.int32),
            pltpu.VMEM((b2,), jnp.int32),
            pltpu.VMEM((b2,), jnp.int32),
            pltpu.VMEM((b2,), jnp.int32),
            pltpu.VMEM((b2, D_H), jnp.float32),
            pltpu.VMEM((b2, D_H), jnp.float32),
            pltpu.VMEM((16,), jnp.int32),
            pltpu.VMEM_SHARED((N_HOST, D_H), jnp.float32),
            pltpu.VMEM_SHARED((CHUNK + 8, D_H), jnp.float32),
            pltpu.SemaphoreType.DMA,
            pltpu.SemaphoreType.DMA,
            pltpu.SemaphoreType.DMA,
            pltpu.SemaphoreType.DMA,
            pltpu.SemaphoreType.DMA,
            pltpu.SemaphoreType.DMA,
        ],
        compiler_params=pltpu.CompilerParams(use_tc_tiling_on_sc=False,
                                             needs_layout_passes=False),
    )


def _sc_agg_flow(h_tab, bsrc, bdst, bcnt):
    z64 = jnp.zeros((CHUNK + 8, D_H), jnp.float32)
    return _make_sc_agg_flow()(h_tab, bsrc, bdst, bcnt, z64)


def _sc_agg_flow(h_tab, bsrc, bdst, bcnt):
    z64 = jnp.zeros((CHUNK + 8, D_H), jnp.float32)
    return _make_sc_agg_flow()(h_tab, bsrc, bdst, bcnt, z64)


def _sc_cnt_flow_body(bdst, bcnt, zc, ones_hbm, out_cnt,
                      dst_v, ones_v, bcnt_v, cnt_sh, sem, sem2):
    c = lax.axis_index("c")
    s = lax.axis_index("s")
    wid = s * _NC + c
    iota = lax.iota(jnp.int32, 16)
    b2c = 512
    pltpu.sync_copy(ones_hbm, ones_v)
    pltpu.sync_copy(bcnt.at[wid], bcnt_v)
    cnt_row = bcnt_v[...]
    zrows = CHUNK // _NS
    for k0 in range(K_CHUNKS):
        pltpu.sync_copy(zc.at[pl.ds(s * zrows, zrows)],
                        cnt_sh.at[pl.ds(s * zrows, zrows)])

        @pl.when(s == 0)
        def _():
            pltpu.sync_copy(zc.at[pl.ds(CHUNK, 8)], cnt_sh.at[pl.ds(CHUNK, 8)])

        plsc.subcore_barrier()
        n16 = cnt_row[k0]
        nfull = lax.shift_right_logical(n16, 5)
        rem = lax.bitwise_and(n16, 31)

        def trip(t, _, k0=k0):
            pltpu.sync_copy(bdst.at[k0, wid, pl.ds(t * b2c, b2c)], dst_v)
            pltpu.sync_copy(ones_v, cnt_sh.at[dst_v], add=True)
            return _

        lax.fori_loop(0, nfull, trip, 0)

        @pl.when(rem > 0)
        def _(k0=k0, nfull=nfull, rem=rem):
            pltpu.sync_copy(bdst.at[k0, wid, pl.ds(nfull * b2c, b2c)], dst_v)

            def fill(j, _):
                dst_v[pl.ds(j * 16, 16)] = jnp.full((16,), _DUMP, jnp.int32)
                return _

            lax.fori_loop(rem, b2c // 16, fill, 0)
            pltpu.sync_copy(ones_v, cnt_sh.at[dst_v], add=True)

        plsc.subcore_barrier()
        pltpu.sync_copy(cnt_sh.at[pl.ds(s * zrows, zrows)],
                        out_cnt.at[c, pl.ds(k0 * CHUNK + s * zrows, zrows)])
        plsc.subcore_barrier()


def _sc_cnt_flow(bdst, bcnt):
    mesh = plsc.VectorSubcoreMesh(core_axis_name="c", subcore_axis_name="s")
    zc = jnp.zeros((CHUNK + 8, _CW), jnp.float32)
    ones = jnp.ones((512, _CW), jnp.float32)
    f = pl.kernel(
        _sc_cnt_flow_body,
        out_type=jax.ShapeDtypeStruct((_NC, P_FLOW, _CW), jnp.float32),
        mesh=mesh,
        scratch_types=[
            pltpu.VMEM((512,), jnp.int32),
            pltpu.VMEM((512, _CW), jnp.float32),
            pltpu.VMEM((16,), jnp.int32),
            pltpu.VMEM_SHARED((CHUNK + 8, _CW), jnp.float32),
            pltpu.SemaphoreType.DMA,
            pltpu.SemaphoreType.DMA,
        ],
        compiler_params=pltpu.CompilerParams(use_tc_tiling_on_sc=False,
                                             needs_layout_passes=False),
    )
    return f(bdst, bcnt, zc, ones)


def kernel(x_host, x_flow, ei_h2f_src, ei_h2f_dst, ei_f2h_src, ei_f2h_dst,
           W_host, b_host, W_flow, b_flow,
           Wl_h2f_0, bl_h2f_0, Wr_h2f_0, Wl_f2h_0, bl_f2h_0, Wr_f2h_0,
           Wl_h2f_1, bl_h2f_1, Wr_h2f_1, Wl_f2h_1, bl_f2h_1, Wr_f2h_1,
           W_out, b_out):
    h0 = _tc_proj_relu(x_host, W_host, b_host, rb=2000)
    f0 = _tc_proj_relu(x_flow, W_flow, b_flow, rb=2000)

    bsrc, bdstl, bcnt, cntf_p = _sc_bucket(ei_h2f_src, ei_h2f_dst)
    aggf0_p = _sc_agg_flow(h0, bsrc, bdstl, bcnt)
    aggh0_p, cnth_p = _sc_agg_host(f0, ei_f2h_src, ei_f2h_dst)

    f = _tc_layer(aggf0_p, cntf_p, f0, Wl_h2f_0, bl_h2f_0, Wr_h2f_0, rb=2000)
    h = _tc_layer(aggh0_p, cnth_p, h0, Wl_f2h_0, bl_f2h_0, Wr_f2h_0, rb=2000)

    aggf1_p = _sc_agg_flow(h, bsrc, bdstl, bcnt)

    return _tc_final(aggf1_p, cntf_p, f, Wl_h2f_1, bl_h2f_1, Wr_h2f_1,
                     W_out, b_out, rb=2000)
